# Initial kernel scaffold; baseline (speedup 1.0000x reference)
import functools

import jax
import jax.numpy as jnp
from jax import lax
from jax.experimental import pallas as pl
from jax.experimental.pallas import tpu as pltpu
from jax.experimental.pallas import tpu_sc as plsc

N = 10000
E = 320000
NPAD = 10240
NC = 2
NS = 16
NW = NC * NS
EPW = E // NW
CHUNK = 80

_MESH = plsc.VectorSubcoreMesh(core_axis_name="c", subcore_axis_name="s")

_ZERO16 = jnp.zeros((16,), jnp.float32)
_ONES16 = jnp.ones((16,), jnp.float32)


def _deg_body(row_hbm, out_hbm, idxbuf, hist, zbuf, iotabuf, acc, sem):
    c = lax.axis_index("c")
    s = lax.axis_index("s")
    w = c * NS + s
    estart = pl.multiple_of(w * EPW, 8)

    def _z(i, _):
        hist[i, :] = _ZERO16
        return 0
    lax.fori_loop(0, 640, _z, 0, unroll=False)

    def _outer(k, _):
        base = pl.multiple_of(estart + k * 2000, 8)
        pltpu.sync_copy(row_hbm.at[pl.ds(base, 2000)], idxbuf)

        def _inner(j, _):
            idx = idxbuf[pl.ds(j * 16, 16)]
            plsc.addupdate_scatter(hist, [idx >> 4, idx & 15], _ONES16)
            return 0
        lax.fori_loop(0, 125, _inner, 0, unroll=False)
        return 0
    lax.fori_loop(0, 5, _outer, 0, unroll=False)

    def _z2(i, _):
        zbuf[i, :] = _ZERO16
        return 0
    lax.fori_loop(0, 40, _z2, 0, unroll=False)
    pltpu.sync_copy(zbuf, acc.at[pl.ds(s * 40, 40)])
    plsc.subcore_barrier()

    for cix in range(5):
        for j in range(8):
            iotabuf[pl.ds(j * 16, 16)] = (
                lax.iota(jnp.int32, (16,)) + (cix * 128 + j * 16))
        pltpu.sync_copy(hist.at[pl.ds(cix * 128, 128)],
                        acc.at[iotabuf], add=True)
    plsc.subcore_barrier()

    pltpu.sync_copy(acc.at[pl.ds(s * 40, 40)],
                    out_hbm.at[c, pl.ds(s * 40, 40)])


_deg_call = pl.kernel(
    _deg_body,
    out_type=jax.ShapeDtypeStruct((NC, 640, 16), jnp.float32),
    mesh=_MESH,
    scratch_types=[
        pltpu.VMEM((2000,), jnp.int32),
        pltpu.VMEM((640, 16), jnp.float32),
        pltpu.VMEM((40, 16), jnp.float32),
        pltpu.VMEM((128,), jnp.int32),
        pltpu.VMEM_SHARED((640, 16), jnp.float32),
        pltpu.SemaphoreType.DMA,
    ],
)


def _spmm_body(x_hbm, row_hbm, col_hbm, out_hbm,
               cbuf, rbuf, rows, zbuf, acc, sem, *, d):
    c = lax.axis_index("c")
    s = lax.axis_index("s")
    w = c * NS + s
    estart = pl.multiple_of(w * EPW, 8)

    def _z(i, _):
        for j in range(d // 16):
            zbuf[i, pl.ds(j * 16, 16)] = _ZERO16
        return 0
    lax.fori_loop(0, 625, _z, 0, unroll=False)
    pltpu.sync_copy(zbuf, acc.at[pl.ds(s * 625, 625)])
    plsc.subcore_barrier()

    def _chunk(g, _):
        base = pl.multiple_of(estart + g * CHUNK, 8)
        pltpu.sync_copy(col_hbm.at[pl.ds(base, CHUNK)], cbuf)
        pltpu.async_copy(x_hbm.at[cbuf], rows, sem).wait()
        pltpu.sync_copy(row_hbm.at[pl.ds(base, CHUNK)], rbuf)
        pltpu.sync_copy(rows, acc.at[rbuf], add=True)
        return 0
    lax.fori_loop(0, EPW // CHUNK, _chunk, 0, unroll=False)
    plsc.subcore_barrier()

    pltpu.sync_copy(acc.at[pl.ds(s * 625, 625)],
                    out_hbm.at[c, pl.ds(s * 625, 625)])


def _make_spmm(d):
    return pl.kernel(
        functools.partial(_spmm_body, d=d),
        out_type=jax.ShapeDtypeStruct((NC, N, d), jnp.float32),
        mesh=_MESH,
        scratch_types=[
            pltpu.VMEM((CHUNK,), jnp.int32),
            pltpu.VMEM((CHUNK,), jnp.int32),
            pltpu.VMEM((CHUNK, d), jnp.float32),
            pltpu.VMEM((625, d), jnp.float32),
            pltpu.VMEM_SHARED((N, d), jnp.float32),
            pltpu.SemaphoreType.DMA,
        ],
    )


_spmm32 = _make_spmm(32)
_spmm16 = _make_spmm(16)


def _enc1_body(h_ref, w_ref, deg_ref, o_ref):
    d = deg_ref[...]
    norm = lax.rsqrt(d[0] + d[1])
    o_ref[...] = jnp.dot(h_ref[...], w_ref[...],
                         preferred_element_type=jnp.float32) * norm


_enc1_call = pl.pallas_call(
    _enc1_body,
    grid=(NPAD // 1024,),
    in_specs=[
        pl.BlockSpec((1024, 128), lambda i: (i, 0)),
        pl.BlockSpec((128, 32), lambda i: (0, 0)),
        pl.BlockSpec((2, 1024, 1), lambda i: (0, i, 0)),
    ],
    out_specs=pl.BlockSpec((1024, 32), lambda i: (i, 0)),
    out_shape=jax.ShapeDtypeStruct((NPAD, 32), jnp.float32),
)


def _enc2_body(p_ref, w_ref, deg_ref, o_ref):
    p = p_ref[...]
    hrelu = jnp.maximum(p[0] + p[1], 0.0)
    d = deg_ref[...]
    inv = 1.0 / (d[0] + d[1])
    o_ref[...] = jnp.dot(hrelu, w_ref[...],
                         preferred_element_type=jnp.float32) * inv


_enc2_call = pl.pallas_call(
    _enc2_body,
    grid=(NPAD // 1024,),
    in_specs=[
        pl.BlockSpec((2, 1024, 32), lambda i: (0, i, 0)),
        pl.BlockSpec((32, 16), lambda i: (0, 0)),
        pl.BlockSpec((2, 1024, 1), lambda i: (0, i, 0)),
    ],
    out_specs=pl.BlockSpec((1024, 16), lambda i: (i, 0)),
    out_shape=jax.ShapeDtypeStruct((NPAD, 16), jnp.float32),
)


def _dec_body(qi_ref, qj_ref, di_ref, dj_ref, o_ref):
    qi = qi_ref[...]
    di = di_ref[...]
    zi = (qi[0] + qi[1]) * lax.rsqrt(di[0] + di[1])
    qj = qj_ref[...]
    dj = dj_ref[...]
    zj = (qj[0] + qj[1]) * lax.rsqrt(dj[0] + dj[1])
    o_ref[...] = lax.dot_general(zi, zj, (((1,), (1,)), ((), ())),
                                 preferred_element_type=jnp.float32)


_BM = 512
_dec_call = pl.pallas_call(
    _dec_body,
    grid=(NPAD // _BM, NPAD // _BM),
    in_specs=[
        pl.BlockSpec((2, _BM, 16), lambda i, j: (0, i, 0)),
        pl.BlockSpec((2, _BM, 16), lambda i, j: (0, j, 0)),
        pl.BlockSpec((2, _BM, 1), lambda i, j: (0, i, 0)),
        pl.BlockSpec((2, _BM, 1), lambda i, j: (0, j, 0)),
    ],
    out_specs=pl.BlockSpec((_BM, _BM), lambda i, j: (i, j)),
    out_shape=jax.ShapeDtypeStruct((N, N), jnp.float32),
    compiler_params=pltpu.CompilerParams(
        dimension_semantics=("parallel", "parallel")),
)


def kernel(h, edge_index, W0, W1):
    row = edge_index[0]
    col = edge_index[1]
    deg_p = _deg_call(row)
    deg2 = deg_p.reshape(NC, NPAD, 1)
    x0 = _enc1_call(h, W0, deg2)
    P = _spmm32(x0, row, col)
    x1 = _enc2_call(P, W1, deg2)
    Q = _spmm16(x1, row, col)
    return _dec_call(Q, Q, deg2, deg2)


# speedup vs baseline: 3.7481x; 3.7481x over previous
import functools

import jax
import jax.numpy as jnp
from jax import lax
from jax.experimental import pallas as pl
from jax.experimental.pallas import tpu as pltpu
from jax.experimental.pallas import tpu_sc as plsc

N = 10000
E = 320000
NPAD = 10240
NC = 2
NS = 16
NW = NC * NS
EPW = E // NW
CHUNK = 80

_MESH = plsc.VectorSubcoreMesh(core_axis_name="c", subcore_axis_name="s")

def _deg_body(row_hbm, out_hbm, idxbuf, hist, tmp, accl, slots, sem):
    _ZERO16 = jnp.zeros((16,), jnp.float32)
    _ONES16 = jnp.ones((16,), jnp.float32)
    c = lax.axis_index("c")
    s = lax.axis_index("s")
    w = c * NS + s
    estart = pl.multiple_of(w * EPW, 8)

    def _z(i, _):
        hist[pl.ds(i * 16, 16)] = _ZERO16
        return 0
    lax.fori_loop(0, 640, _z, 0, unroll=False)

    def _outer(k, _):
        base = pl.multiple_of(estart + k * 2000, 8)
        pltpu.sync_copy(row_hbm.at[pl.ds(base, 2000)], idxbuf)

        def _inner(j, _):
            idx = idxbuf[pl.ds(j * 16, 16)]
            plsc.addupdate_scatter(hist, [idx], _ONES16)
            return 0
        lax.fori_loop(0, 125, _inner, 0, unroll=False)
        return 0
    lax.fori_loop(0, 5, _outer, 0, unroll=False)

    pltpu.sync_copy(hist, slots.at[s])
    plsc.subcore_barrier()

    nbase = s * 640
    pltpu.sync_copy(slots.at[0, pl.ds(nbase, 640)], accl)

    def _red(j, _):
        pltpu.sync_copy(slots.at[j, pl.ds(nbase, 640)], tmp)

        def _add(i, _):
            accl[pl.ds(i * 16, 16)] = (
                accl[pl.ds(i * 16, 16)] + tmp[pl.ds(i * 16, 16)])
            return 0
        lax.fori_loop(0, 40, _add, 0, unroll=False)
        return 0
    lax.fori_loop(1, NS, _red, 0, unroll=False)

    pltpu.sync_copy(accl, out_hbm.at[c, pl.ds(nbase, 640)])


_deg_call = pl.kernel(
    _deg_body,
    out_type=jax.ShapeDtypeStruct((NC, NPAD), jnp.float32),
    mesh=_MESH,
    scratch_types=[
        pltpu.VMEM((2000,), jnp.int32),
        pltpu.VMEM((NPAD,), jnp.float32),
        pltpu.VMEM((640,), jnp.float32),
        pltpu.VMEM((640,), jnp.float32),
        pltpu.VMEM_SHARED((NS, NPAD), jnp.float32),
        pltpu.SemaphoreType.DMA,
    ],
    compiler_params=pltpu.CompilerParams(needs_layout_passes=False),
)


def _spmm_body(x_hbm, row_hbm, col_hbm, out_hbm,
               cbuf, rbuf, rows, zbuf, acc, sem, *, d):
    _ZERO16 = jnp.zeros((16,), jnp.float32)
    c = lax.axis_index("c")
    s = lax.axis_index("s")
    w = c * NS + s
    estart = pl.multiple_of(w * EPW, 8)

    def _z(i, _):
        for j in range(d // 16):
            zbuf[i, pl.ds(j * 16, 16)] = _ZERO16
        return 0
    lax.fori_loop(0, 640, _z, 0, unroll=False)
    pltpu.sync_copy(zbuf, acc.at[pl.ds(s * 640, 640)])
    plsc.subcore_barrier()

    def _chunk(g, _):
        base = pl.multiple_of(estart + g * CHUNK, 8)
        pltpu.sync_copy(col_hbm.at[pl.ds(base, CHUNK)], cbuf)
        pltpu.async_copy(x_hbm.at[cbuf], rows, sem).wait()
        pltpu.sync_copy(row_hbm.at[pl.ds(base, CHUNK)], rbuf)
        pltpu.sync_copy(rows, acc.at[rbuf], add=True)
        return 0
    lax.fori_loop(0, EPW // CHUNK, _chunk, 0, unroll=False)
    plsc.subcore_barrier()

    pltpu.sync_copy(acc.at[pl.ds(s * 640, 640)],
                    out_hbm.at[c, pl.ds(s * 640, 640)])


def _make_spmm(d):
    return pl.kernel(
        functools.partial(_spmm_body, d=d),
        out_type=jax.ShapeDtypeStruct((NC, NPAD, d), jnp.float32),
        mesh=_MESH,
        scratch_types=[
            pltpu.VMEM((CHUNK,), jnp.int32),
            pltpu.VMEM((CHUNK,), jnp.int32),
            pltpu.VMEM((CHUNK, d), jnp.float32),
            pltpu.VMEM((640, d), jnp.float32),
            pltpu.VMEM_SHARED((NPAD, d), jnp.float32),
            pltpu.SemaphoreType.DMA,
        ],
        compiler_params=pltpu.CompilerParams(
            needs_layout_passes=False, use_tc_tiling_on_sc=False),
    )


_spmm32 = _make_spmm(32)
_spmm16 = _make_spmm(16)


def _enc1_body(h_ref, w_ref, deg_ref, o_ref):
    d = deg_ref[...]
    norm = lax.rsqrt(d[0] + d[1])
    o_ref[...] = jnp.dot(h_ref[...], w_ref[...],
                         preferred_element_type=jnp.float32) * norm


_enc1_call = pl.pallas_call(
    _enc1_body,
    grid=(NPAD // 1024,),
    in_specs=[
        pl.BlockSpec((1024, 128), lambda i: (i, 0)),
        pl.BlockSpec((128, 32), lambda i: (0, 0)),
        pl.BlockSpec((2, 1024, 1), lambda i: (0, i, 0)),
    ],
    out_specs=pl.BlockSpec((1024, 32), lambda i: (i, 0)),
    out_shape=jax.ShapeDtypeStruct((NPAD, 32), jnp.float32),
)


def _enc2_body(p_ref, w_ref, deg_ref, o_ref):
    p = p_ref[...]
    hrelu = jnp.maximum(p[0] + p[1], 0.0)
    d = deg_ref[...]
    inv = 1.0 / (d[0] + d[1])
    o_ref[...] = jnp.dot(hrelu, w_ref[...],
                         preferred_element_type=jnp.float32) * inv


_enc2_call = pl.pallas_call(
    _enc2_body,
    grid=(NPAD // 1024,),
    in_specs=[
        pl.BlockSpec((2, 1024, 32), lambda i: (0, i, 0)),
        pl.BlockSpec((32, 16), lambda i: (0, 0)),
        pl.BlockSpec((2, 1024, 1), lambda i: (0, i, 0)),
    ],
    out_specs=pl.BlockSpec((1024, 16), lambda i: (i, 0)),
    out_shape=jax.ShapeDtypeStruct((NPAD, 16), jnp.float32),
)


def _dec_body(qi_ref, qj_ref, di_ref, dj_ref, o_ref):
    qi = qi_ref[...]
    di = di_ref[...]
    zi = (qi[0] + qi[1]) * lax.rsqrt(di[0] + di[1])
    qj = qj_ref[...]
    dj = dj_ref[...]
    zj = (qj[0] + qj[1]) * lax.rsqrt(dj[0] + dj[1])
    o_ref[...] = lax.dot_general(zi, zj, (((1,), (1,)), ((), ())),
                                 preferred_element_type=jnp.float32)


_BM = 512
_dec_call = pl.pallas_call(
    _dec_body,
    grid=(NPAD // _BM, NPAD // _BM),
    in_specs=[
        pl.BlockSpec((2, _BM, 16), lambda i, j: (0, i, 0)),
        pl.BlockSpec((2, _BM, 16), lambda i, j: (0, j, 0)),
        pl.BlockSpec((2, _BM, 1), lambda i, j: (0, i, 0)),
        pl.BlockSpec((2, _BM, 1), lambda i, j: (0, j, 0)),
    ],
    out_specs=pl.BlockSpec((_BM, _BM), lambda i, j: (i, j)),
    out_shape=jax.ShapeDtypeStruct((N, N), jnp.float32),
    compiler_params=pltpu.CompilerParams(
        dimension_semantics=("parallel", "parallel")),
)


def kernel(h, edge_index, W0, W1):
    row = edge_index[0]
    col = edge_index[1]
    deg_p = _deg_call(row)
    deg2 = deg_p.reshape(NC, NPAD, 1)
    x0 = _enc1_call(h, W0, deg2)
    P = _spmm32(x0, row, col)
    x1 = _enc2_call(P, W1, deg2)
    Q = _spmm16(x1, row, col)
    return _dec_call(Q, Q, deg2, deg2)


# speedup vs baseline: 4.9276x; 1.3147x over previous
import functools

import jax
import jax.numpy as jnp
from jax import lax
from jax.experimental import pallas as pl
from jax.experimental.pallas import tpu as pltpu
from jax.experimental.pallas import tpu_sc as plsc

N = 10000
E = 320000
NPAD = 10240
NC = 2
NS = 16
NW = NC * NS
EPW = E // NW
CHUNK = 80

_MESH = plsc.VectorSubcoreMesh(core_axis_name="c", subcore_axis_name="s")

def _deg_body(row_hbm, out_hbm, idxbuf, hist, tmp, accl, slots, sem):
    _ZERO16 = jnp.zeros((16,), jnp.float32)
    _ONES16 = jnp.ones((16,), jnp.float32)
    c = lax.axis_index("c")
    s = lax.axis_index("s")
    w = c * NS + s
    estart = pl.multiple_of(w * EPW, 8)

    def _z(i, _):
        hist[pl.ds(i * 16, 16)] = _ZERO16
        return 0
    lax.fori_loop(0, 640, _z, 0, unroll=False)

    def _outer(k, _):
        base = pl.multiple_of(estart + k * 2000, 8)
        pltpu.sync_copy(row_hbm.at[pl.ds(base, 2000)], idxbuf)

        def _inner(j, _):
            idx = idxbuf[pl.ds(j * 16, 16)]
            plsc.addupdate_scatter(hist, [idx], _ONES16)
            return 0
        lax.fori_loop(0, 125, _inner, 0, unroll=False)
        return 0
    lax.fori_loop(0, 5, _outer, 0, unroll=False)

    pltpu.sync_copy(hist, slots.at[s])
    plsc.subcore_barrier()

    nbase = s * 640
    pltpu.sync_copy(slots.at[0, pl.ds(nbase, 640)], accl)

    def _red(j, _):
        pltpu.sync_copy(slots.at[j, pl.ds(nbase, 640)], tmp)

        def _add(i, _):
            accl[pl.ds(i * 16, 16)] = (
                accl[pl.ds(i * 16, 16)] + tmp[pl.ds(i * 16, 16)])
            return 0
        lax.fori_loop(0, 40, _add, 0, unroll=False)
        return 0
    lax.fori_loop(1, NS, _red, 0, unroll=False)

    pltpu.sync_copy(accl, out_hbm.at[c, pl.ds(nbase, 640)])


_deg_call = pl.kernel(
    _deg_body,
    out_type=jax.ShapeDtypeStruct((NC, NPAD), jnp.float32),
    mesh=_MESH,
    scratch_types=[
        pltpu.VMEM((2000,), jnp.int32),
        pltpu.VMEM((NPAD,), jnp.float32),
        pltpu.VMEM((640,), jnp.float32),
        pltpu.VMEM((640,), jnp.float32),
        pltpu.VMEM_SHARED((NS, NPAD), jnp.float32),
        pltpu.SemaphoreType.DMA,
    ],
    compiler_params=pltpu.CompilerParams(needs_layout_passes=False),
)


NCHUNK = EPW // CHUNK


def _spmm_body(x_hbm, row2_hbm, col2_hbm, out_hbm,
               cstage, rstage, rows0, rows1, zbuf, acc,
               semg0, semg1, sems0, sems1, *, d):
    _ZERO16 = jnp.zeros((16,), jnp.float32)
    c = lax.axis_index("c")
    s = lax.axis_index("s")
    w = c * NS + s
    cbase = w * NCHUNK

    pltpu.sync_copy(col2_hbm.at[pl.ds(cbase, NCHUNK)], cstage)
    pltpu.sync_copy(row2_hbm.at[pl.ds(cbase, NCHUNK)], rstage)

    def _z(i, _):
        for j in range(d // 16):
            zbuf[i, pl.ds(j * 16, 16)] = _ZERO16
        return 0
    lax.fori_loop(0, 640, _z, 0, unroll=False)
    pltpu.sync_copy(zbuf, acc.at[pl.ds(s * 640, 640)])
    plsc.subcore_barrier()

    def _gather(j, buf, sem):
        pltpu.async_copy(x_hbm.at[cstage.at[j]], buf, sem)

    def _wait_gather(buf, sem):
        pltpu.make_async_copy(x_hbm.at[cstage.at[0]], buf, sem).wait()

    def _scatter(j, buf, sem):
        pltpu.async_copy(buf, acc.at[rstage.at[j]], sem, add=True)

    def _wait_scatter(buf, sem):
        pltpu.make_async_copy(buf, acc.at[rstage.at[0]], sem).wait()

    _gather(0, rows0, semg0)

    def _pair(t, _):
        e = 2 * t
        _wait_gather(rows0, semg0)

        @pl.when(t > 0)
        def _():
            _wait_scatter(rows1, sems1)

        _gather(e + 1, rows1, semg1)
        _scatter(e, rows0, sems0)
        _wait_gather(rows1, semg1)
        _wait_scatter(rows0, sems0)
        _gather(e + 2, rows0, semg0)
        _scatter(e + 1, rows1, sems1)
        return 0
    lax.fori_loop(0, (NCHUNK - 1) // 2, _pair, 0, unroll=False)

    _wait_gather(rows0, semg0)
    _wait_scatter(rows1, sems1)
    _scatter(NCHUNK - 1, rows0, sems0)
    _wait_scatter(rows0, sems0)
    plsc.subcore_barrier()

    pltpu.sync_copy(acc.at[pl.ds(s * 640, 640)],
                    out_hbm.at[c, pl.ds(s * 640, 640)])


def _make_spmm(d):
    return pl.kernel(
        functools.partial(_spmm_body, d=d),
        out_type=jax.ShapeDtypeStruct((NC, NPAD, d), jnp.float32),
        mesh=_MESH,
        scratch_types=[
            pltpu.VMEM((NCHUNK, CHUNK), jnp.int32),
            pltpu.VMEM((NCHUNK, CHUNK), jnp.int32),
            pltpu.VMEM((CHUNK, d), jnp.float32),
            pltpu.VMEM((CHUNK, d), jnp.float32),
            pltpu.VMEM((640, d), jnp.float32),
            pltpu.VMEM_SHARED((NPAD, d), jnp.float32),
            pltpu.SemaphoreType.DMA,
            pltpu.SemaphoreType.DMA,
            pltpu.SemaphoreType.DMA,
            pltpu.SemaphoreType.DMA,
        ],
        compiler_params=pltpu.CompilerParams(
            needs_layout_passes=False, use_tc_tiling_on_sc=False),
    )


_spmm32 = _make_spmm(32)
_spmm16 = _make_spmm(16)


def _enc1_body(h_ref, w_ref, deg_ref, o_ref):
    d = deg_ref[...]
    norm = lax.rsqrt(d[0] + d[1])
    o_ref[...] = jnp.dot(h_ref[...], w_ref[...],
                         preferred_element_type=jnp.float32) * norm


_enc1_call = pl.pallas_call(
    _enc1_body,
    grid=(NPAD // 1024,),
    in_specs=[
        pl.BlockSpec((1024, 128), lambda i: (i, 0)),
        pl.BlockSpec((128, 32), lambda i: (0, 0)),
        pl.BlockSpec((2, 1024, 1), lambda i: (0, i, 0)),
    ],
    out_specs=pl.BlockSpec((1024, 32), lambda i: (i, 0)),
    out_shape=jax.ShapeDtypeStruct((NPAD, 32), jnp.float32),
)


def _enc2_body(p_ref, w_ref, deg_ref, o_ref):
    p = p_ref[...]
    hrelu = jnp.maximum(p[0] + p[1], 0.0)
    d = deg_ref[...]
    inv = 1.0 / (d[0] + d[1])
    o_ref[...] = jnp.dot(hrelu, w_ref[...],
                         preferred_element_type=jnp.float32) * inv


_enc2_call = pl.pallas_call(
    _enc2_body,
    grid=(NPAD // 1024,),
    in_specs=[
        pl.BlockSpec((2, 1024, 32), lambda i: (0, i, 0)),
        pl.BlockSpec((32, 16), lambda i: (0, 0)),
        pl.BlockSpec((2, 1024, 1), lambda i: (0, i, 0)),
    ],
    out_specs=pl.BlockSpec((1024, 16), lambda i: (i, 0)),
    out_shape=jax.ShapeDtypeStruct((NPAD, 16), jnp.float32),
)


def _dec_body(qi_ref, qj_ref, di_ref, dj_ref, o_ref):
    qi = qi_ref[...]
    di = di_ref[...]
    zi = (qi[0] + qi[1]) * lax.rsqrt(di[0] + di[1])
    qj = qj_ref[...]
    dj = dj_ref[...]
    zj = (qj[0] + qj[1]) * lax.rsqrt(dj[0] + dj[1])
    o_ref[...] = lax.dot_general(zi, zj, (((1,), (1,)), ((), ())),
                                 preferred_element_type=jnp.float32)


_BM = 512
_dec_call = pl.pallas_call(
    _dec_body,
    grid=(NPAD // _BM, NPAD // _BM),
    in_specs=[
        pl.BlockSpec((2, _BM, 16), lambda i, j: (0, i, 0)),
        pl.BlockSpec((2, _BM, 16), lambda i, j: (0, j, 0)),
        pl.BlockSpec((2, _BM, 1), lambda i, j: (0, i, 0)),
        pl.BlockSpec((2, _BM, 1), lambda i, j: (0, j, 0)),
    ],
    out_specs=pl.BlockSpec((_BM, _BM), lambda i, j: (i, j)),
    out_shape=jax.ShapeDtypeStruct((N, N), jnp.float32),
    compiler_params=pltpu.CompilerParams(
        dimension_semantics=("parallel", "parallel")),
)


def kernel(h, edge_index, W0, W1):
    row = edge_index[0]
    col = edge_index[1]
    row2 = row.reshape(E // CHUNK, CHUNK)
    col2 = col.reshape(E // CHUNK, CHUNK)
    deg_p = _deg_call(row)
    deg2 = deg_p.reshape(NC, NPAD, 1)
    x0 = _enc1_call(h, W0, deg2)
    P = _spmm32(x0, row2, col2)
    x1 = _enc2_call(P, W1, deg2)
    Q = _spmm16(x1, row2, col2)
    return _dec_call(Q, Q, deg2, deg2)


# speedup vs baseline: 6.6689x; 1.3534x over previous
import functools

import jax
import jax.numpy as jnp
from jax import lax
from jax.experimental import pallas as pl
from jax.experimental.pallas import tpu as pltpu
from jax.experimental.pallas import tpu_sc as plsc

N = 10000
E = 320000
NPAD = 10240
NC = 2
NS = 16
NW = NC * NS
EPW = E // NW
CHUNK = 80

_MESH = plsc.VectorSubcoreMesh(core_axis_name="c", subcore_axis_name="s")

def _deg_body(ei_hbm, out_hbm, idxbuf, hist, tmp, accl, slots, sem):
    _ZERO16 = jnp.zeros((16,), jnp.float32)
    _ONES16 = jnp.ones((16,), jnp.float32)
    c = lax.axis_index("c")
    s = lax.axis_index("s")
    w = c * NS + s
    estart = pl.multiple_of(w * EPW, 8)

    def _z(i, _):
        hist[pl.ds(i * 16, 16)] = _ZERO16
        return 0
    lax.fori_loop(0, 640, _z, 0, unroll=False)

    def _outer(k, _):
        base = pl.multiple_of(estart + k * 2000, 8)
        pltpu.sync_copy(ei_hbm.at[0, pl.ds(base, 2000)], idxbuf)

        def _inner(j, _):
            idx = idxbuf[pl.ds(j * 16, 16)]
            plsc.addupdate_scatter(hist, [idx], _ONES16)
            return 0
        lax.fori_loop(0, 125, _inner, 0, unroll=False)
        return 0
    lax.fori_loop(0, 5, _outer, 0, unroll=False)

    pltpu.sync_copy(hist, slots.at[s])
    plsc.subcore_barrier()

    nbase = s * 640
    pltpu.sync_copy(slots.at[0, pl.ds(nbase, 640)], accl)

    def _red(j, _):
        pltpu.sync_copy(slots.at[j, pl.ds(nbase, 640)], tmp)

        def _add(i, _):
            accl[pl.ds(i * 16, 16)] = (
                accl[pl.ds(i * 16, 16)] + tmp[pl.ds(i * 16, 16)])
            return 0
        lax.fori_loop(0, 40, _add, 0, unroll=False)
        return 0
    lax.fori_loop(1, NS, _red, 0, unroll=False)

    pltpu.sync_copy(accl, out_hbm.at[c, pl.ds(nbase, 640)])


_deg_call = pl.kernel(
    _deg_body,
    out_type=jax.ShapeDtypeStruct((NC, NPAD), jnp.float32),
    mesh=_MESH,
    scratch_types=[
        pltpu.VMEM((2000,), jnp.int32),
        pltpu.VMEM((NPAD,), jnp.float32),
        pltpu.VMEM((640,), jnp.float32),
        pltpu.VMEM((640,), jnp.float32),
        pltpu.VMEM_SHARED((NS, NPAD), jnp.float32),
        pltpu.SemaphoreType.DMA,
    ],
    compiler_params=pltpu.CompilerParams(
        needs_layout_passes=False, use_tc_tiling_on_sc=False),
)


NCHUNK = EPW // CHUNK


def _spmm_body(x_hbm, ei3_hbm, out_hbm,
               cstage, rstage, rows0, rows1, zbuf, acc,
               semg0, semg1, sems0, sems1, *, d):
    _ZERO16 = jnp.zeros((16,), jnp.float32)
    c = lax.axis_index("c")
    s = lax.axis_index("s")
    w = c * NS + s
    cbase = w * NCHUNK

    pltpu.sync_copy(ei3_hbm.at[1, pl.ds(cbase, NCHUNK)], cstage)
    pltpu.sync_copy(ei3_hbm.at[0, pl.ds(cbase, NCHUNK)], rstage)

    def _z(i, _):
        for j in range(d // 16):
            zbuf[i, pl.ds(j * 16, 16)] = _ZERO16
        return 0
    lax.fori_loop(0, 640, _z, 0, unroll=False)
    pltpu.sync_copy(zbuf, acc.at[pl.ds(s * 640, 640)])
    plsc.subcore_barrier()

    def _gather(j, buf, sem):
        pltpu.async_copy(x_hbm.at[cstage.at[j]], buf, sem)

    def _wait_gather(buf, sem):
        pltpu.make_async_copy(x_hbm.at[cstage.at[0]], buf, sem).wait()

    def _scatter(j, buf, sem):
        pltpu.async_copy(buf, acc.at[rstage.at[j]], sem, add=True)

    def _wait_scatter(buf, sem):
        pltpu.make_async_copy(buf, acc.at[rstage.at[0]], sem).wait()

    _gather(0, rows0, semg0)

    def _pair(t, _):
        e = 2 * t
        _wait_gather(rows0, semg0)

        @pl.when(t > 0)
        def _():
            _wait_scatter(rows1, sems1)

        _gather(e + 1, rows1, semg1)
        _scatter(e, rows0, sems0)
        _wait_gather(rows1, semg1)
        _wait_scatter(rows0, sems0)
        _gather(e + 2, rows0, semg0)
        _scatter(e + 1, rows1, sems1)
        return 0
    lax.fori_loop(0, (NCHUNK - 1) // 2, _pair, 0, unroll=False)

    _wait_gather(rows0, semg0)
    _wait_scatter(rows1, sems1)
    _scatter(NCHUNK - 1, rows0, sems0)
    _wait_scatter(rows0, sems0)
    plsc.subcore_barrier()

    pltpu.sync_copy(acc.at[pl.ds(s * 640, 640)],
                    out_hbm.at[c, pl.ds(s * 640, 640)])


def _make_spmm(d):
    return pl.kernel(
        functools.partial(_spmm_body, d=d),
        out_type=jax.ShapeDtypeStruct((NC, NPAD, d), jnp.float32),
        mesh=_MESH,
        scratch_types=[
            pltpu.VMEM((NCHUNK, CHUNK), jnp.int32),
            pltpu.VMEM((NCHUNK, CHUNK), jnp.int32),
            pltpu.VMEM((CHUNK, d), jnp.float32),
            pltpu.VMEM((CHUNK, d), jnp.float32),
            pltpu.VMEM((640, d), jnp.float32),
            pltpu.VMEM_SHARED((NPAD, d), jnp.float32),
            pltpu.SemaphoreType.DMA,
            pltpu.SemaphoreType.DMA,
            pltpu.SemaphoreType.DMA,
            pltpu.SemaphoreType.DMA,
        ],
        compiler_params=pltpu.CompilerParams(
            needs_layout_passes=False, use_tc_tiling_on_sc=False),
    )


_spmm32 = _make_spmm(32)
_spmm16 = _make_spmm(16)


def _enc1_body(h_ref, w_ref, deg_ref, o_ref):
    d = deg_ref[...]
    norm = lax.rsqrt(d[0] + d[1])
    o_ref[...] = jnp.dot(h_ref[...], w_ref[...],
                         preferred_element_type=jnp.float32) * norm


_enc1_call = pl.pallas_call(
    _enc1_body,
    grid=(NPAD // 1024,),
    in_specs=[
        pl.BlockSpec((1024, 128), lambda i: (i, 0)),
        pl.BlockSpec((128, 32), lambda i: (0, 0)),
        pl.BlockSpec((2, 1024, 1), lambda i: (0, i, 0)),
    ],
    out_specs=pl.BlockSpec((1024, 32), lambda i: (i, 0)),
    out_shape=jax.ShapeDtypeStruct((NPAD, 32), jnp.float32),
)


def _enc2_body(p_ref, w_ref, deg_ref, o_ref):
    p = p_ref[...]
    hrelu = jnp.maximum(p[0] + p[1], 0.0)
    d = deg_ref[...]
    inv = 1.0 / (d[0] + d[1])
    o_ref[...] = jnp.dot(hrelu, w_ref[...],
                         preferred_element_type=jnp.float32) * inv


_enc2_call = pl.pallas_call(
    _enc2_body,
    grid=(NPAD // 1024,),
    in_specs=[
        pl.BlockSpec((2, 1024, 32), lambda i: (0, i, 0)),
        pl.BlockSpec((32, 16), lambda i: (0, 0)),
        pl.BlockSpec((2, 1024, 1), lambda i: (0, i, 0)),
    ],
    out_specs=pl.BlockSpec((1024, 16), lambda i: (i, 0)),
    out_shape=jax.ShapeDtypeStruct((NPAD, 16), jnp.float32),
)


def _dec_body(qi_ref, qj_ref, di_ref, dj_ref, o_ref):
    qi = qi_ref[...]
    di = di_ref[...]
    zi = (qi[0] + qi[1]) * lax.rsqrt(di[0] + di[1])
    qj = qj_ref[...]
    dj = dj_ref[...]
    zj = (qj[0] + qj[1]) * lax.rsqrt(dj[0] + dj[1])
    o_ref[...] = lax.dot_general(zi, zj, (((1,), (1,)), ((), ())),
                                 preferred_element_type=jnp.float32)


_BM = 512
_BN = 2048
_dec_call = pl.pallas_call(
    _dec_body,
    grid=(NPAD // _BM, NPAD // _BN),
    in_specs=[
        pl.BlockSpec((2, _BM, 16), lambda i, j: (0, i, 0)),
        pl.BlockSpec((2, _BN, 16), lambda i, j: (0, j, 0)),
        pl.BlockSpec((2, _BM, 1), lambda i, j: (0, i, 0)),
        pl.BlockSpec((2, _BN, 1), lambda i, j: (0, j, 0)),
    ],
    out_specs=pl.BlockSpec((_BM, _BN), lambda i, j: (i, j)),
    out_shape=jax.ShapeDtypeStruct((N, N), jnp.float32),
    compiler_params=pltpu.CompilerParams(
        dimension_semantics=("parallel", "parallel")),
)


def kernel(h, edge_index, W0, W1):
    ei3 = edge_index.reshape(2, E // CHUNK, CHUNK)
    deg_p = _deg_call(edge_index)
    deg2 = deg_p.reshape(NC, NPAD, 1)
    x0 = _enc1_call(h, W0, deg2)
    P = _spmm32(x0, ei3)
    Q = _spmm16(_enc2_call(P, W1, deg2), ei3)
    return _dec_call(Q, Q, deg2, deg2)


# speedup vs baseline: 7.8944x; 1.1838x over previous
import functools

import jax
import jax.numpy as jnp
from jax import lax
from jax.experimental import pallas as pl
from jax.experimental.pallas import tpu as pltpu
from jax.experimental.pallas import tpu_sc as plsc

N = 10000
E = 320000
NPAD = 10240
NC = 2
NS = 16
NW = NC * NS
EPW = E // NW
CHUNK = 80

_MESH = plsc.VectorSubcoreMesh(core_axis_name="c", subcore_axis_name="s")

def _deg_body(ei_hbm, out_hbm, idxbuf, hist, tmp, accl, slots, sem):
    _ZERO16 = jnp.zeros((16,), jnp.float32)
    _ONES16 = jnp.ones((16,), jnp.float32)
    c = lax.axis_index("c")
    s = lax.axis_index("s")
    w = c * NS + s
    estart = pl.multiple_of(w * EPW, 8)

    def _z(i, _):
        hist[pl.ds(i * 16, 16)] = _ZERO16
        return 0
    lax.fori_loop(0, 640, _z, 0, unroll=False)

    def _outer(k, _):
        base = pl.multiple_of(estart + k * 2000, 8)
        pltpu.sync_copy(ei_hbm.at[0, pl.ds(base, 2000)], idxbuf)

        def _inner(j, _):
            idx = idxbuf[pl.ds(j * 16, 16)]
            plsc.addupdate_scatter(hist, [idx], _ONES16)
            return 0
        lax.fori_loop(0, 125, _inner, 0, unroll=False)
        return 0
    lax.fori_loop(0, 5, _outer, 0, unroll=False)

    pltpu.sync_copy(hist, slots.at[s])
    plsc.subcore_barrier()

    nbase = s * 640
    pltpu.sync_copy(slots.at[0, pl.ds(nbase, 640)], accl)

    def _red(j, _):
        pltpu.sync_copy(slots.at[j, pl.ds(nbase, 640)], tmp)

        def _add(i, _):
            accl[pl.ds(i * 16, 16)] = (
                accl[pl.ds(i * 16, 16)] + tmp[pl.ds(i * 16, 16)])
            return 0
        lax.fori_loop(0, 40, _add, 0, unroll=False)
        return 0
    lax.fori_loop(1, NS, _red, 0, unroll=False)

    pltpu.sync_copy(accl, out_hbm.at[c, pl.ds(nbase, 640)])


_deg_call = pl.kernel(
    _deg_body,
    out_type=jax.ShapeDtypeStruct((NC, NPAD), jnp.float32),
    mesh=_MESH,
    scratch_types=[
        pltpu.VMEM((2000,), jnp.int32),
        pltpu.VMEM((NPAD,), jnp.float32),
        pltpu.VMEM((640,), jnp.float32),
        pltpu.VMEM((640,), jnp.float32),
        pltpu.VMEM_SHARED((NS, NPAD), jnp.float32),
        pltpu.SemaphoreType.DMA,
    ],
    compiler_params=pltpu.CompilerParams(
        needs_layout_passes=False, use_tc_tiling_on_sc=False),
)


NCHUNK = EPW // CHUNK


def _spmm_body(x_hbm, ei3_hbm, out_hbm,
               cstage, rstage, b0, b1, b2, b3, b4, zbuf, acc,
               g0, g1, g2, g3, g4, s0, s1, s2, s3, s4, stsem, *, d):
    _ZERO16 = jnp.zeros((16,), jnp.float32)
    c = lax.axis_index("c")
    s = lax.axis_index("s")
    w = c * NS + s
    cbase = w * NCHUNK
    bufs = (b0, b1, b2, b3, b4)
    gsems = (g0, g1, g2, g3, g4)
    ssems = (s0, s1, s2, s3, s4)

    pltpu.async_copy(ei3_hbm.at[1, pl.ds(cbase, NCHUNK)], cstage, stsem)
    pltpu.async_copy(ei3_hbm.at[0, pl.ds(cbase, NCHUNK)], rstage, stsem)

    def _z(i, _):
        for j in range(d // 16):
            zbuf[i, pl.ds(j * 16, 16)] = _ZERO16
        return 0
    lax.fori_loop(0, 640, _z, 0, unroll=False)
    pltpu.sync_copy(zbuf, acc.at[pl.ds(s * 640, 640)])
    pltpu.make_async_copy(ei3_hbm.at[1, pl.ds(cbase, NCHUNK)], cstage,
                          stsem).wait()
    pltpu.make_async_copy(ei3_hbm.at[0, pl.ds(cbase, NCHUNK)], rstage,
                          stsem).wait()
    plsc.subcore_barrier()

    def _gather(j, buf, sem):
        pltpu.async_copy(x_hbm.at[cstage.at[j]], buf, sem)

    def _wait_gather(buf, sem):
        pltpu.make_async_copy(x_hbm.at[cstage.at[0]], buf, sem).wait()

    def _scatter(j, buf, sem):
        pltpu.async_copy(buf, acc.at[rstage.at[j]], sem, add=True)

    def _wait_scatter(buf, sem):
        pltpu.make_async_copy(buf, acc.at[rstage.at[0]], sem).wait()

    _gather(0, bufs[0], gsems[0])
    _gather(1, bufs[1], gsems[1])

    def _iter(t, _):
        q0 = 5 * t
        for b in range(5):
            q = q0 + b
            n = (b + 2) % 5
            _wait_gather(bufs[b], gsems[b])

            @pl.when(q >= 3)
            def _():
                _wait_scatter(bufs[n], ssems[n])

            @pl.when(q <= NCHUNK - 3)
            def _():
                _gather(q + 2, bufs[n], gsems[n])

            _scatter(q, bufs[b], ssems[b])
        return 0
    lax.fori_loop(0, NCHUNK // 5, _iter, 0, unroll=False)
    for b in (2, 3, 4):
        _wait_scatter(bufs[b], ssems[b])
    plsc.subcore_barrier()

    pltpu.sync_copy(acc.at[pl.ds(s * 640, 640)],
                    out_hbm.at[c, pl.ds(s * 640, 640)])


def _make_spmm(d):
    return pl.kernel(
        functools.partial(_spmm_body, d=d),
        out_type=jax.ShapeDtypeStruct((NC, NPAD, d), jnp.float32),
        mesh=_MESH,
        scratch_types=[
            pltpu.VMEM((NCHUNK, CHUNK), jnp.int32),
            pltpu.VMEM((NCHUNK, CHUNK), jnp.int32),
        ] + [pltpu.VMEM((CHUNK, d), jnp.float32)] * 5 + [
            pltpu.VMEM((640, d), jnp.float32),
            pltpu.VMEM_SHARED((NPAD, d), jnp.float32),
        ] + [pltpu.SemaphoreType.DMA] * 11,
        compiler_params=pltpu.CompilerParams(
            needs_layout_passes=False, use_tc_tiling_on_sc=False),
    )


_spmm32 = _make_spmm(32)
_spmm16 = _make_spmm(16)


def _enc1_body(h_ref, w_ref, deg_ref, o_ref):
    d = deg_ref[...]
    norm = lax.rsqrt(d[0] + d[1])
    o_ref[...] = jnp.dot(h_ref[...], w_ref[...],
                         preferred_element_type=jnp.float32) * norm


_enc1_call = pl.pallas_call(
    _enc1_body,
    grid=(NPAD // 1024,),
    in_specs=[
        pl.BlockSpec((1024, 128), lambda i: (i, 0)),
        pl.BlockSpec((128, 32), lambda i: (0, 0)),
        pl.BlockSpec((2, 1024, 1), lambda i: (0, i, 0)),
    ],
    out_specs=pl.BlockSpec((1024, 32), lambda i: (i, 0)),
    out_shape=jax.ShapeDtypeStruct((NPAD, 32), jnp.float32),
)


def _enc2_body(p_ref, w_ref, deg_ref, o_ref):
    p = p_ref[...]
    hrelu = jnp.maximum(p[0] + p[1], 0.0)
    d = deg_ref[...]
    inv = 1.0 / (d[0] + d[1])
    o_ref[...] = jnp.dot(hrelu, w_ref[...],
                         preferred_element_type=jnp.float32) * inv


_enc2_call = pl.pallas_call(
    _enc2_body,
    grid=(NPAD // 1024,),
    in_specs=[
        pl.BlockSpec((2, 1024, 32), lambda i: (0, i, 0)),
        pl.BlockSpec((32, 16), lambda i: (0, 0)),
        pl.BlockSpec((2, 1024, 1), lambda i: (0, i, 0)),
    ],
    out_specs=pl.BlockSpec((1024, 16), lambda i: (i, 0)),
    out_shape=jax.ShapeDtypeStruct((NPAD, 16), jnp.float32),
)


def _dec_body(qi_ref, qj_ref, di_ref, dj_ref, o_ref):
    qi = qi_ref[...]
    di = di_ref[...]
    zi = (qi[0] + qi[1]) * lax.rsqrt(di[0] + di[1])
    qj = qj_ref[...]
    dj = dj_ref[...]
    zj = (qj[0] + qj[1]) * lax.rsqrt(dj[0] + dj[1])
    o_ref[...] = lax.dot_general(zi, zj, (((1,), (1,)), ((), ())),
                                 preferred_element_type=jnp.float32)


_BM = 512
_BN = 2048
_dec_call = pl.pallas_call(
    _dec_body,
    grid=(NPAD // _BM, NPAD // _BN),
    in_specs=[
        pl.BlockSpec((2, _BM, 16), lambda i, j: (0, i, 0)),
        pl.BlockSpec((2, _BN, 16), lambda i, j: (0, j, 0)),
        pl.BlockSpec((2, _BM, 1), lambda i, j: (0, i, 0)),
        pl.BlockSpec((2, _BN, 1), lambda i, j: (0, j, 0)),
    ],
    out_specs=pl.BlockSpec((_BM, _BN), lambda i, j: (i, j)),
    out_shape=jax.ShapeDtypeStruct((N, N), jnp.float32),
    compiler_params=pltpu.CompilerParams(
        dimension_semantics=("parallel", "parallel")),
)


def kernel(h, edge_index, W0, W1):
    ei3 = edge_index.reshape(2, E // CHUNK, CHUNK)
    deg_p = _deg_call(edge_index)
    deg2 = deg_p.reshape(NC, NPAD, 1)
    x0 = _enc1_call(h, W0, deg2)
    P = _spmm32(x0, ei3)
    Q = _spmm16(_enc2_call(P, W1, deg2), ei3)
    return _dec_call(Q, Q, deg2, deg2)


# speedup vs baseline: 8.2936x; 1.0506x over previous
import functools

import jax
import jax.numpy as jnp
from jax import lax
from jax.experimental import pallas as pl
from jax.experimental.pallas import tpu as pltpu
from jax.experimental.pallas import tpu_sc as plsc

N = 10000
E = 320000
NPAD = 10240
NC = 2
NS = 16
NW = NC * NS
EPW = E // NW
CHUNK = 125

_MESH = plsc.VectorSubcoreMesh(core_axis_name="c", subcore_axis_name="s")

def _deg_body(ei_hbm, out_hbm, idxbuf, hist, tmp, accl, slots, sem):
    _ZERO16 = jnp.zeros((16,), jnp.float32)
    _ONES16 = jnp.ones((16,), jnp.float32)
    c = lax.axis_index("c")
    s = lax.axis_index("s")
    w = c * NS + s
    estart = pl.multiple_of(w * EPW, 8)

    def _z(i, _):
        hist[pl.ds(i * 16, 16)] = _ZERO16
        return 0
    lax.fori_loop(0, 640, _z, 0, unroll=False)

    def _outer(k, _):
        base = pl.multiple_of(estart + k * 2000, 8)
        pltpu.sync_copy(ei_hbm.at[0, pl.ds(base, 2000)], idxbuf)

        def _inner(j, _):
            idx = idxbuf[pl.ds(j * 16, 16)]
            plsc.addupdate_scatter(hist, [idx], _ONES16)
            return 0
        lax.fori_loop(0, 125, _inner, 0, unroll=False)
        return 0
    lax.fori_loop(0, 5, _outer, 0, unroll=False)

    pltpu.sync_copy(hist, slots.at[s])
    plsc.subcore_barrier()

    nbase = s * 640
    pltpu.sync_copy(slots.at[0, pl.ds(nbase, 640)], accl)

    def _red(j, _):
        pltpu.sync_copy(slots.at[j, pl.ds(nbase, 640)], tmp)

        def _add(i, _):
            accl[pl.ds(i * 16, 16)] = (
                accl[pl.ds(i * 16, 16)] + tmp[pl.ds(i * 16, 16)])
            return 0
        lax.fori_loop(0, 40, _add, 0, unroll=False)
        return 0
    lax.fori_loop(1, NS, _red, 0, unroll=False)

    pltpu.sync_copy(accl, out_hbm.at[c, pl.ds(nbase, 640)])


_deg_call = pl.kernel(
    _deg_body,
    out_type=jax.ShapeDtypeStruct((NC, NPAD), jnp.float32),
    mesh=_MESH,
    scratch_types=[
        pltpu.VMEM((2000,), jnp.int32),
        pltpu.VMEM((NPAD,), jnp.float32),
        pltpu.VMEM((640,), jnp.float32),
        pltpu.VMEM((640,), jnp.float32),
        pltpu.VMEM_SHARED((NS, NPAD), jnp.float32),
        pltpu.SemaphoreType.DMA,
    ],
    compiler_params=pltpu.CompilerParams(
        needs_layout_passes=False, use_tc_tiling_on_sc=False),
)


NCHUNK = EPW // CHUNK


def _spmm_body(x_hbm, ei3_hbm, out_hbm,
               cstage, rstage, b0, b1, b2, b3, b4, zbuf, acc,
               g0, g1, g2, g3, g4, s0, s1, s2, s3, s4, stsem, *, d):
    _ZERO16 = jnp.zeros((16,), jnp.float32)
    c = lax.axis_index("c")
    s = lax.axis_index("s")
    w = c * NS + s
    cbase = w * NCHUNK
    bufs = (b0, b1, b2, b3, b4)
    gsems = (g0, g1, g2, g3, g4)
    ssems = (s0, s1, s2, s3, s4)

    pltpu.async_copy(ei3_hbm.at[1, pl.ds(cbase, NCHUNK)], cstage, stsem)
    pltpu.async_copy(ei3_hbm.at[0, pl.ds(cbase, NCHUNK)], rstage, stsem)

    def _z(i, _):
        for j in range(d // 16):
            zbuf[i, pl.ds(j * 16, 16)] = _ZERO16
        return 0
    lax.fori_loop(0, 640, _z, 0, unroll=False)
    pltpu.sync_copy(zbuf, acc.at[pl.ds(s * 640, 640)])
    pltpu.make_async_copy(ei3_hbm.at[1, pl.ds(cbase, NCHUNK)], cstage,
                          stsem).wait()
    pltpu.make_async_copy(ei3_hbm.at[0, pl.ds(cbase, NCHUNK)], rstage,
                          stsem).wait()
    plsc.subcore_barrier()

    def _gather(j, buf, sem):
        pltpu.async_copy(x_hbm.at[cstage.at[j]], buf, sem)

    def _wait_gather(buf, sem):
        pltpu.make_async_copy(x_hbm.at[cstage.at[0]], buf, sem).wait()

    def _scatter(j, buf, sem):
        pltpu.async_copy(buf, acc.at[rstage.at[j]], sem, add=True)

    def _wait_scatter(buf, sem):
        pltpu.make_async_copy(buf, acc.at[rstage.at[0]], sem).wait()

    _gather(0, bufs[0], gsems[0])
    _gather(1, bufs[1], gsems[1])

    def _iter(t, _):
        q0 = 5 * t
        for b in range(5):
            q = q0 + b
            n = (b + 2) % 5
            _wait_gather(bufs[b], gsems[b])

            @pl.when(q >= 3)
            def _():
                _wait_scatter(bufs[n], ssems[n])

            @pl.when(q <= NCHUNK - 3)
            def _():
                _gather(q + 2, bufs[n], gsems[n])

            _scatter(q, bufs[b], ssems[b])
        return 0
    lax.fori_loop(0, NCHUNK // 5, _iter, 0, unroll=False)
    for b in (2, 3, 4):
        _wait_scatter(bufs[b], ssems[b])
    plsc.subcore_barrier()

    pltpu.sync_copy(acc.at[pl.ds(s * 640, 640)],
                    out_hbm.at[c, pl.ds(s * 640, 640)])


def _make_spmm(d):
    return pl.kernel(
        functools.partial(_spmm_body, d=d),
        out_type=jax.ShapeDtypeStruct((NC, NPAD, d), jnp.float32),
        mesh=_MESH,
        scratch_types=[
            pltpu.VMEM((NCHUNK, CHUNK), jnp.int32),
            pltpu.VMEM((NCHUNK, CHUNK), jnp.int32),
        ] + [pltpu.VMEM((CHUNK, d), jnp.float32)] * 5 + [
            pltpu.VMEM((640, d), jnp.float32),
            pltpu.VMEM_SHARED((NPAD, d), jnp.float32),
        ] + [pltpu.SemaphoreType.DMA] * 11,
        compiler_params=pltpu.CompilerParams(
            needs_layout_passes=False, use_tc_tiling_on_sc=False),
    )


_spmm32 = _make_spmm(32)
_spmm16 = _make_spmm(16)


def _enc1_body(h_ref, w_ref, deg_ref, o_ref):
    d = deg_ref[...]
    norm = lax.rsqrt(d[0] + d[1])
    o_ref[...] = jnp.dot(h_ref[...], w_ref[...],
                         preferred_element_type=jnp.float32) * norm


_enc1_call = pl.pallas_call(
    _enc1_body,
    grid=(NPAD // 1024,),
    in_specs=[
        pl.BlockSpec((1024, 128), lambda i: (i, 0)),
        pl.BlockSpec((128, 32), lambda i: (0, 0)),
        pl.BlockSpec((2, 1024, 1), lambda i: (0, i, 0)),
    ],
    out_specs=pl.BlockSpec((1024, 32), lambda i: (i, 0)),
    out_shape=jax.ShapeDtypeStruct((NPAD, 32), jnp.float32),
)


def _enc2_body(p_ref, w_ref, deg_ref, o_ref):
    p = p_ref[...]
    hrelu = jnp.maximum(p[0] + p[1], 0.0)
    d = deg_ref[...]
    inv = 1.0 / (d[0] + d[1])
    o_ref[...] = jnp.dot(hrelu, w_ref[...],
                         preferred_element_type=jnp.float32) * inv


_enc2_call = pl.pallas_call(
    _enc2_body,
    grid=(NPAD // 1024,),
    in_specs=[
        pl.BlockSpec((2, 1024, 32), lambda i: (0, i, 0)),
        pl.BlockSpec((32, 16), lambda i: (0, 0)),
        pl.BlockSpec((2, 1024, 1), lambda i: (0, i, 0)),
    ],
    out_specs=pl.BlockSpec((1024, 16), lambda i: (i, 0)),
    out_shape=jax.ShapeDtypeStruct((NPAD, 16), jnp.float32),
)


def _dec_body(qi_ref, qj_ref, di_ref, dj_ref, o_ref):
    qi = qi_ref[...]
    di = di_ref[...]
    zi = (qi[0] + qi[1]) * lax.rsqrt(di[0] + di[1])
    qj = qj_ref[...]
    dj = dj_ref[...]
    zj = (qj[0] + qj[1]) * lax.rsqrt(dj[0] + dj[1])
    o_ref[...] = lax.dot_general(zi, zj, (((1,), (1,)), ((), ())),
                                 preferred_element_type=jnp.float32)


_BM = 512
_BN = 2048
_dec_call = pl.pallas_call(
    _dec_body,
    grid=(NPAD // _BM, NPAD // _BN),
    in_specs=[
        pl.BlockSpec((2, _BM, 16), lambda i, j: (0, i, 0)),
        pl.BlockSpec((2, _BN, 16), lambda i, j: (0, j, 0)),
        pl.BlockSpec((2, _BM, 1), lambda i, j: (0, i, 0)),
        pl.BlockSpec((2, _BN, 1), lambda i, j: (0, j, 0)),
    ],
    out_specs=pl.BlockSpec((_BM, _BN), lambda i, j: (i, j)),
    out_shape=jax.ShapeDtypeStruct((N, N), jnp.float32),
    compiler_params=pltpu.CompilerParams(
        dimension_semantics=("parallel", "parallel")),
)


def kernel(h, edge_index, W0, W1):
    ei3 = edge_index.reshape(2, E // CHUNK, CHUNK)
    deg_p = _deg_call(edge_index)
    deg2 = deg_p.reshape(NC, NPAD, 1)
    x0 = _enc1_call(h, W0, deg2)
    P = _spmm32(x0, ei3)
    Q = _spmm16(_enc2_call(P, W1, deg2), ei3)
    return _dec_call(Q, Q, deg2, deg2)


# speedup vs baseline: 9.7673x; 1.1777x over previous
import functools

import jax
import jax.numpy as jnp
from jax import lax
from jax.experimental import pallas as pl
from jax.experimental.pallas import tpu as pltpu
from jax.experimental.pallas import tpu_sc as plsc

N = 10000
E = 320000
NPAD = 10240
NC = 2
NS = 16
NW = NC * NS
EPW = E // NW
CHUNK = 125

_MESH = plsc.VectorSubcoreMesh(core_axis_name="c", subcore_axis_name="s")

def _deg_body(ei_hbm, out_hbm, idxbuf, hist, tmp, accl, slots, sem):
    _ZERO16 = jnp.zeros((16,), jnp.float32)
    _ONES16 = jnp.ones((16,), jnp.float32)
    c = lax.axis_index("c")
    s = lax.axis_index("s")
    w = c * NS + s
    estart = pl.multiple_of(w * EPW, 8)

    def _z(i, _):
        hist[pl.ds(i * 16, 16)] = _ZERO16
        return 0
    lax.fori_loop(0, 640, _z, 0, unroll=False)

    def _outer(k, _):
        base = pl.multiple_of(estart + k * 2000, 8)
        pltpu.sync_copy(ei_hbm.at[0, pl.ds(base, 2000)], idxbuf)

        def _inner(j, _):
            idx = idxbuf[pl.ds(j * 16, 16)]
            plsc.addupdate_scatter(hist, [idx], _ONES16)
            return 0
        lax.fori_loop(0, 125, _inner, 0, unroll=False)
        return 0
    lax.fori_loop(0, 5, _outer, 0, unroll=False)

    pltpu.sync_copy(hist, slots.at[s])
    plsc.subcore_barrier()

    nbase = s * 640
    pltpu.sync_copy(slots.at[0, pl.ds(nbase, 640)], accl)

    def _red(j, _):
        pltpu.sync_copy(slots.at[j, pl.ds(nbase, 640)], tmp)

        def _add(i, _):
            accl[pl.ds(i * 16, 16)] = (
                accl[pl.ds(i * 16, 16)] + tmp[pl.ds(i * 16, 16)])
            return 0
        lax.fori_loop(0, 40, _add, 0, unroll=False)
        return 0
    lax.fori_loop(1, NS, _red, 0, unroll=False)

    pltpu.sync_copy(accl, out_hbm.at[c, pl.ds(nbase, 640)])


_deg_call = pl.kernel(
    _deg_body,
    out_type=jax.ShapeDtypeStruct((NC, NPAD), jnp.float32),
    mesh=_MESH,
    scratch_types=[
        pltpu.VMEM((2000,), jnp.int32),
        pltpu.VMEM((NPAD,), jnp.float32),
        pltpu.VMEM((640,), jnp.float32),
        pltpu.VMEM((640,), jnp.float32),
        pltpu.VMEM_SHARED((NS, NPAD), jnp.float32),
        pltpu.SemaphoreType.DMA,
    ],
    compiler_params=pltpu.CompilerParams(
        needs_layout_passes=False, use_tc_tiling_on_sc=False),
)


NCHUNK = EPW // CHUNK


def _spmm_body(x_hbm, ei3_hbm, out_hbm,
               cstage, rstage, b0, b1, b2, b3, b4, zbuf, acc,
               g0, g1, g2, g3, g4, s0, s1, s2, s3, s4, stsem, *, d):
    _ZERO16 = jnp.zeros((16,), jnp.float32)
    c = lax.axis_index("c")
    s = lax.axis_index("s")
    w = c * NS + s
    cbase = w * NCHUNK
    bufs = (b0, b1, b2, b3, b4)
    gsems = (g0, g1, g2, g3, g4)
    ssems = (s0, s1, s2, s3, s4)

    pltpu.async_copy(ei3_hbm.at[1, pl.ds(cbase, NCHUNK)], cstage, stsem)
    pltpu.async_copy(ei3_hbm.at[0, pl.ds(cbase, NCHUNK)], rstage, stsem)

    def _z(i, _):
        for j in range(d // 16):
            zbuf[i, pl.ds(j * 16, 16)] = _ZERO16
        return 0
    lax.fori_loop(0, 640, _z, 0, unroll=False)
    pltpu.sync_copy(zbuf, acc.at[pl.ds(s * 640, 640)])
    pltpu.make_async_copy(ei3_hbm.at[1, pl.ds(cbase, NCHUNK)], cstage,
                          stsem).wait()
    pltpu.make_async_copy(ei3_hbm.at[0, pl.ds(cbase, NCHUNK)], rstage,
                          stsem).wait()
    plsc.subcore_barrier()

    def _gather(j, buf, sem):
        pltpu.async_copy(x_hbm.at[cstage.at[j]], buf, sem)

    def _wait_gather(buf, sem):
        pltpu.make_async_copy(x_hbm.at[cstage.at[0]], buf, sem).wait()

    def _scatter(j, buf, sem):
        pltpu.async_copy(buf, acc.at[rstage.at[j]], sem, add=True)

    def _wait_scatter(buf, sem):
        pltpu.make_async_copy(buf, acc.at[rstage.at[0]], sem).wait()

    _gather(0, bufs[0], gsems[0])
    _gather(1, bufs[1], gsems[1])

    def _iter(t, _):
        q0 = 5 * t
        for b in range(5):
            q = q0 + b
            n = (b + 2) % 5
            _wait_gather(bufs[b], gsems[b])

            @pl.when(q >= 3)
            def _():
                _wait_scatter(bufs[n], ssems[n])

            @pl.when(q <= NCHUNK - 3)
            def _():
                _gather(q + 2, bufs[n], gsems[n])

            _scatter(q, bufs[b], ssems[b])
        return 0
    lax.fori_loop(0, NCHUNK // 5, _iter, 0, unroll=False)
    for b in (2, 3, 4):
        _wait_scatter(bufs[b], ssems[b])
    plsc.subcore_barrier()

    pltpu.sync_copy(acc.at[pl.ds(s * 640, 640)],
                    out_hbm.at[c, pl.ds(s * 640, 640)])


def _make_spmm(d):
    return pl.kernel(
        functools.partial(_spmm_body, d=d),
        out_type=jax.ShapeDtypeStruct((NC, NPAD, d), jnp.float32),
        mesh=_MESH,
        scratch_types=[
            pltpu.VMEM((NCHUNK, CHUNK), jnp.int32),
            pltpu.VMEM((NCHUNK, CHUNK), jnp.int32),
        ] + [pltpu.VMEM((CHUNK, d), jnp.float32)] * 5 + [
            pltpu.VMEM((640, d), jnp.float32),
            pltpu.VMEM_SHARED((NPAD, d), jnp.float32),
        ] + [pltpu.SemaphoreType.DMA] * 11,
        compiler_params=pltpu.CompilerParams(
            needs_layout_passes=False, use_tc_tiling_on_sc=False),
    )


_spmm32 = _make_spmm(32)
_spmm16 = _make_spmm(16)


def _enc1_body(h_ref, w_ref, deg_ref, o_ref):
    d = deg_ref[...]
    norm = lax.rsqrt(d[0] + d[1])
    o_ref[...] = jnp.dot(h_ref[...], w_ref[...],
                         preferred_element_type=jnp.float32) * norm


_enc1_call = pl.pallas_call(
    _enc1_body,
    grid=(NPAD // 1024,),
    in_specs=[
        pl.BlockSpec((1024, 128), lambda i: (i, 0)),
        pl.BlockSpec((128, 32), lambda i: (0, 0)),
        pl.BlockSpec((2, 1024, 1), lambda i: (0, i, 0)),
    ],
    out_specs=pl.BlockSpec((1024, 32), lambda i: (i, 0)),
    out_shape=jax.ShapeDtypeStruct((NPAD, 32), jnp.float32),
)


def _enc2_body(p_ref, w_ref, deg_ref, o_ref):
    p = p_ref[...]
    hrelu = jnp.maximum(p[0] + p[1], 0.0)
    d = deg_ref[...]
    inv = 1.0 / (d[0] + d[1])
    o_ref[...] = jnp.dot(hrelu, w_ref[...],
                         preferred_element_type=jnp.float32) * inv


_enc2_call = pl.pallas_call(
    _enc2_body,
    grid=(NPAD // 1024,),
    in_specs=[
        pl.BlockSpec((2, 1024, 32), lambda i: (0, i, 0)),
        pl.BlockSpec((32, 16), lambda i: (0, 0)),
        pl.BlockSpec((2, 1024, 1), lambda i: (0, i, 0)),
    ],
    out_specs=pl.BlockSpec((1024, 16), lambda i: (i, 0)),
    out_shape=jax.ShapeDtypeStruct((NPAD, 16), jnp.float32),
)


def _dec_body(qi_ref, qj_ref, di_ref, dj_ref, o_ref):
    qi = qi_ref[...]
    di = di_ref[...]
    zi = (qi[0] + qi[1]) * lax.rsqrt(di[0] + di[1])
    qj = qj_ref[...]
    dj = dj_ref[...]
    zj = (qj[0] + qj[1]) * lax.rsqrt(dj[0] + dj[1])
    o_ref[...] = lax.dot_general(zi, zj, (((1,), (1,)), ((), ())),
                                 preferred_element_type=jnp.float32)


_BM = 1024
_BN = 2048
_dec_call = pl.pallas_call(
    _dec_body,
    grid=(NPAD // _BM, NPAD // _BN),
    in_specs=[
        pl.BlockSpec((2, _BM, 16), lambda i, j: (0, i, 0)),
        pl.BlockSpec((2, _BN, 16), lambda i, j: (0, j, 0)),
        pl.BlockSpec((2, _BM, 1), lambda i, j: (0, i, 0)),
        pl.BlockSpec((2, _BN, 1), lambda i, j: (0, j, 0)),
    ],
    out_specs=pl.BlockSpec((_BM, _BN), lambda i, j: (i, j)),
    out_shape=jax.ShapeDtypeStruct((N, N), jnp.float32),
    compiler_params=pltpu.CompilerParams(
        dimension_semantics=("parallel", "parallel")),
)


def kernel(h, edge_index, W0, W1):
    ei3 = edge_index.reshape(2, E // CHUNK, CHUNK)
    deg_p = _deg_call(edge_index)
    deg2 = deg_p.reshape(NC, NPAD, 1)
    x0 = _enc1_call(h, W0, deg2)
    P = _spmm32(x0, ei3)
    Q = _spmm16(_enc2_call(P, W1, deg2), ei3)
    return _dec_call(Q, Q, deg2, deg2)


# speedup vs baseline: 10.8905x; 1.1150x over previous
import functools

import jax
import jax.numpy as jnp
from jax import lax
from jax.experimental import pallas as pl
from jax.experimental.pallas import tpu as pltpu
from jax.experimental.pallas import tpu_sc as plsc

N = 10000
E = 320000
NPAD = 10240
NC = 2
NS = 16
NW = NC * NS
EPW = E // NW
CHUNK = 125

_MESH = plsc.VectorSubcoreMesh(core_axis_name="c", subcore_axis_name="s")

def _deg_body(ei_hbm, out_hbm, idxbuf, hist, tmp, accl, slots, sem):
    _ZERO16 = jnp.zeros((16,), jnp.float32)
    _ONES16 = jnp.ones((16,), jnp.float32)
    c = lax.axis_index("c")
    s = lax.axis_index("s")
    w = c * NS + s
    estart = pl.multiple_of(w * EPW, 8)

    def _z(i, _):
        hist[pl.ds(i * 16, 16)] = _ZERO16
        return 0
    lax.fori_loop(0, 640, _z, 0, unroll=False)

    def _outer(k, _):
        base = pl.multiple_of(estart + k * 2000, 8)
        pltpu.sync_copy(ei_hbm.at[0, pl.ds(base, 2000)], idxbuf)

        def _inner(j, _):
            idx = idxbuf[pl.ds(j * 16, 16)]
            plsc.addupdate_scatter(hist, [idx], _ONES16)
            return 0
        lax.fori_loop(0, 125, _inner, 0, unroll=False)
        return 0
    lax.fori_loop(0, 5, _outer, 0, unroll=False)

    pltpu.sync_copy(hist, slots.at[s])
    plsc.subcore_barrier()

    nbase = s * 640
    pltpu.sync_copy(slots.at[0, pl.ds(nbase, 640)], accl)

    def _red(j, _):
        pltpu.sync_copy(slots.at[j, pl.ds(nbase, 640)], tmp)

        def _add(i, _):
            accl[pl.ds(i * 16, 16)] = (
                accl[pl.ds(i * 16, 16)] + tmp[pl.ds(i * 16, 16)])
            return 0
        lax.fori_loop(0, 40, _add, 0, unroll=False)
        return 0
    lax.fori_loop(1, NS, _red, 0, unroll=False)

    pltpu.sync_copy(accl, out_hbm.at[c, pl.ds(nbase, 640)])


_deg_call = pl.kernel(
    _deg_body,
    out_type=jax.ShapeDtypeStruct((NC, NPAD), jnp.float32),
    mesh=_MESH,
    scratch_types=[
        pltpu.VMEM((2000,), jnp.int32),
        pltpu.VMEM((NPAD,), jnp.float32),
        pltpu.VMEM((640,), jnp.float32),
        pltpu.VMEM((640,), jnp.float32),
        pltpu.VMEM_SHARED((NS, NPAD), jnp.float32),
        pltpu.SemaphoreType.DMA,
    ],
    compiler_params=pltpu.CompilerParams(
        needs_layout_passes=False, use_tc_tiling_on_sc=False),
)


NCHUNK = EPW // CHUNK


def _spmm_body(x_hbm, ei3_hbm, out_hbm,
               cstage, rstage, b0, b1, b2, b3, b4, zbuf, acc,
               g0, g1, g2, g3, g4, s0, s1, s2, s3, s4, stsem, *, d):
    _ZERO16 = jnp.zeros((16,), jnp.float32)
    c = lax.axis_index("c")
    s = lax.axis_index("s")
    w = c * NS + s
    cbase = w * NCHUNK
    bufs = (b0, b1, b2, b3, b4)
    gsems = (g0, g1, g2, g3, g4)
    ssems = (s0, s1, s2, s3, s4)

    pltpu.async_copy(ei3_hbm.at[1, pl.ds(cbase, NCHUNK)], cstage, stsem)
    pltpu.async_copy(ei3_hbm.at[0, pl.ds(cbase, NCHUNK)], rstage, stsem)

    def _z(i, _):
        for j in range(d // 16):
            zbuf[i, pl.ds(j * 16, 16)] = _ZERO16
        return 0
    lax.fori_loop(0, 640, _z, 0, unroll=False)
    pltpu.sync_copy(zbuf, acc.at[pl.ds(s * 640, 640)])
    pltpu.make_async_copy(ei3_hbm.at[1, pl.ds(cbase, NCHUNK)], cstage,
                          stsem).wait()
    pltpu.make_async_copy(ei3_hbm.at[0, pl.ds(cbase, NCHUNK)], rstage,
                          stsem).wait()
    plsc.subcore_barrier()

    def _gather(j, buf, sem):
        pltpu.async_copy(x_hbm.at[cstage.at[j]], buf, sem)

    def _wait_gather(buf, sem):
        pltpu.make_async_copy(x_hbm.at[cstage.at[0]], buf, sem).wait()

    def _scatter(j, buf, sem):
        pltpu.async_copy(buf, acc.at[rstage.at[j]], sem, add=True)

    def _wait_scatter(buf, sem):
        pltpu.make_async_copy(buf, acc.at[rstage.at[0]], sem).wait()

    _gather(0, bufs[0], gsems[0])
    _gather(1, bufs[1], gsems[1])

    def _iter(t, _):
        q0 = 5 * t
        for b in range(5):
            q = q0 + b
            n = (b + 2) % 5
            _wait_gather(bufs[b], gsems[b])

            @pl.when(q >= 3)
            def _():
                _wait_scatter(bufs[n], ssems[n])

            @pl.when(q <= NCHUNK - 3)
            def _():
                _gather(q + 2, bufs[n], gsems[n])

            _scatter(q, bufs[b], ssems[b])
        return 0
    lax.fori_loop(0, NCHUNK // 5, _iter, 0, unroll=False)
    for b in (2, 3, 4):
        _wait_scatter(bufs[b], ssems[b])
    plsc.subcore_barrier()

    pltpu.sync_copy(acc.at[pl.ds(s * 640, 640)],
                    out_hbm.at[c, pl.ds(s * 640, 640)])


def _make_spmm(d):
    return pl.kernel(
        functools.partial(_spmm_body, d=d),
        out_type=jax.ShapeDtypeStruct((NC, NPAD, d), jnp.float32),
        mesh=_MESH,
        scratch_types=[
            pltpu.VMEM((NCHUNK, CHUNK), jnp.int32),
            pltpu.VMEM((NCHUNK, CHUNK), jnp.int32),
        ] + [pltpu.VMEM((CHUNK, d), jnp.float32)] * 5 + [
            pltpu.VMEM((640, d), jnp.float32),
            pltpu.VMEM_SHARED((NPAD, d), jnp.float32),
        ] + [pltpu.SemaphoreType.DMA] * 11,
        compiler_params=pltpu.CompilerParams(
            needs_layout_passes=False, use_tc_tiling_on_sc=False),
    )


_spmm32 = _make_spmm(32)
_spmm16 = _make_spmm(16)


def _enc1_body(h_ref, w_ref, deg_ref, o_ref):
    d = deg_ref[...]
    norm = lax.rsqrt(d[0] + d[1])
    o_ref[...] = jnp.dot(h_ref[...], w_ref[...],
                         preferred_element_type=jnp.float32) * norm


_enc1_call = pl.pallas_call(
    _enc1_body,
    grid=(NPAD // 1024,),
    in_specs=[
        pl.BlockSpec((1024, 128), lambda i: (i, 0)),
        pl.BlockSpec((128, 32), lambda i: (0, 0)),
        pl.BlockSpec((2, 1024, 1), lambda i: (0, i, 0)),
    ],
    out_specs=pl.BlockSpec((1024, 32), lambda i: (i, 0)),
    out_shape=jax.ShapeDtypeStruct((NPAD, 32), jnp.float32),
)


def _enc2_body(p_ref, w_ref, deg_ref, o_ref):
    p = p_ref[...]
    hrelu = jnp.maximum(p[0] + p[1], 0.0)
    d = deg_ref[...]
    inv = 1.0 / (d[0] + d[1])
    o_ref[...] = jnp.dot(hrelu, w_ref[...],
                         preferred_element_type=jnp.float32) * inv


_enc2_call = pl.pallas_call(
    _enc2_body,
    grid=(NPAD // 1024,),
    in_specs=[
        pl.BlockSpec((2, 1024, 32), lambda i: (0, i, 0)),
        pl.BlockSpec((32, 16), lambda i: (0, 0)),
        pl.BlockSpec((2, 1024, 1), lambda i: (0, i, 0)),
    ],
    out_specs=pl.BlockSpec((1024, 16), lambda i: (i, 0)),
    out_shape=jax.ShapeDtypeStruct((NPAD, 16), jnp.float32),
)


def _dec_body(qi_ref, qj_ref, di_ref, dj_ref, o_ref):
    qi = qi_ref[...]
    di = di_ref[...]
    zi = (qi[0] + qi[1]) * lax.rsqrt(di[0] + di[1])
    qj = qj_ref[...]
    dj = dj_ref[...]
    zj = (qj[0] + qj[1]) * lax.rsqrt(dj[0] + dj[1])
    o_ref[...] = lax.dot_general(zi, zj, (((1,), (1,)), ((), ())),
                                 preferred_element_type=jnp.float32)


_BM = 2048
_BN = 2048
_dec_call = pl.pallas_call(
    _dec_body,
    grid=(NPAD // _BM, NPAD // _BN),
    in_specs=[
        pl.BlockSpec((2, _BM, 16), lambda i, j: (0, i, 0)),
        pl.BlockSpec((2, _BN, 16), lambda i, j: (0, j, 0)),
        pl.BlockSpec((2, _BM, 1), lambda i, j: (0, i, 0)),
        pl.BlockSpec((2, _BN, 1), lambda i, j: (0, j, 0)),
    ],
    out_specs=pl.BlockSpec((_BM, _BN), lambda i, j: (i, j)),
    out_shape=jax.ShapeDtypeStruct((N, N), jnp.float32),
    compiler_params=pltpu.CompilerParams(
        dimension_semantics=("parallel", "parallel")),
)


def kernel(h, edge_index, W0, W1):
    ei3 = edge_index.reshape(2, E // CHUNK, CHUNK)
    deg_p = _deg_call(edge_index)
    deg2 = deg_p.reshape(NC, NPAD, 1)
    x0 = _enc1_call(h, W0, deg2)
    P = _spmm32(x0, ei3)
    Q = _spmm16(_enc2_call(P, W1, deg2), ei3)
    return _dec_call(Q, Q, deg2, deg2)


# speedup vs baseline: 11.4270x; 1.0493x over previous
import functools

import jax
import jax.numpy as jnp
from jax import lax
from jax.experimental import pallas as pl
from jax.experimental.pallas import tpu as pltpu
from jax.experimental.pallas import tpu_sc as plsc

N = 10000
E = 320000
NPAD = 10240
NC = 2
NS = 16
NW = NC * NS
EPW = E // NW
CHUNK = 125

_MESH = plsc.VectorSubcoreMesh(core_axis_name="c", subcore_axis_name="s")

def _deg_body(ei_hbm, out_hbm, idxbuf, hist, tmp, accl, slots, sem):
    _ZERO16 = jnp.zeros((16,), jnp.float32)
    _ONES16 = jnp.ones((16,), jnp.float32)
    c = lax.axis_index("c")
    s = lax.axis_index("s")
    w = c * NS + s
    estart = pl.multiple_of(w * EPW, 8)

    def _z(i, _):
        hist[pl.ds(i * 16, 16)] = _ZERO16
        return 0
    lax.fori_loop(0, 640, _z, 0, unroll=False)

    def _outer(k, _):
        base = pl.multiple_of(estart + k * 2000, 8)
        pltpu.sync_copy(ei_hbm.at[0, pl.ds(base, 2000)], idxbuf)

        def _inner(j, _):
            idx = idxbuf[pl.ds(j * 16, 16)]
            plsc.addupdate_scatter(hist, [idx], _ONES16)
            return 0
        lax.fori_loop(0, 125, _inner, 0, unroll=False)
        return 0
    lax.fori_loop(0, 5, _outer, 0, unroll=False)

    pltpu.sync_copy(hist, slots.at[s])
    plsc.subcore_barrier()

    nbase = s * 640
    pltpu.sync_copy(slots.at[0, pl.ds(nbase, 640)], accl)

    def _red(j, _):
        pltpu.sync_copy(slots.at[j, pl.ds(nbase, 640)], tmp)

        def _add(i, _):
            accl[pl.ds(i * 16, 16)] = (
                accl[pl.ds(i * 16, 16)] + tmp[pl.ds(i * 16, 16)])
            return 0
        lax.fori_loop(0, 40, _add, 0, unroll=False)
        return 0
    lax.fori_loop(1, NS, _red, 0, unroll=False)

    pltpu.sync_copy(accl, out_hbm.at[c, pl.ds(nbase, 640)])


_deg_call = pl.kernel(
    _deg_body,
    out_type=jax.ShapeDtypeStruct((NC, NPAD), jnp.float32),
    mesh=_MESH,
    scratch_types=[
        pltpu.VMEM((2000,), jnp.int32),
        pltpu.VMEM((NPAD,), jnp.float32),
        pltpu.VMEM((640,), jnp.float32),
        pltpu.VMEM((640,), jnp.float32),
        pltpu.VMEM_SHARED((NS, NPAD), jnp.float32),
        pltpu.SemaphoreType.DMA,
    ],
    compiler_params=pltpu.CompilerParams(
        needs_layout_passes=False, use_tc_tiling_on_sc=False),
)


NCHUNK = EPW // CHUNK
NBUF = 8
LOOK = 3


def _spmm_body(x_hbm, ei3_hbm, out_hbm,
               cstage, rstage, bufs, zbuf, acc, gsems, ssems, stsem, *, d):
    _ZERO16 = jnp.zeros((16,), jnp.float32)
    c = lax.axis_index("c")
    s = lax.axis_index("s")
    w = c * NS + s
    cbase = w * NCHUNK

    pltpu.async_copy(ei3_hbm.at[1, pl.ds(cbase, NCHUNK)], cstage, stsem)
    pltpu.async_copy(ei3_hbm.at[0, pl.ds(cbase, NCHUNK)], rstage, stsem)

    def _z(i, _):
        for j in range(d // 16):
            zbuf[i, pl.ds(j * 16, 16)] = _ZERO16
        return 0
    lax.fori_loop(0, 640, _z, 0, unroll=False)
    pltpu.sync_copy(zbuf, acc.at[pl.ds(s * 640, 640)])
    pltpu.make_async_copy(ei3_hbm.at[1, pl.ds(cbase, NCHUNK)], cstage,
                          stsem).wait()
    pltpu.make_async_copy(ei3_hbm.at[0, pl.ds(cbase, NCHUNK)], rstage,
                          stsem).wait()
    plsc.subcore_barrier()

    def _gather(j, b):
        pltpu.async_copy(x_hbm.at[cstage.at[j]], bufs[b], gsems[b])

    def _wait_gather(b):
        pltpu.make_async_copy(x_hbm.at[cstage.at[0]], bufs[b], gsems[b]).wait()

    def _scatter(j, b):
        pltpu.async_copy(bufs[b], acc.at[rstage.at[j]], ssems[b], add=True)

    def _wait_scatter(b):
        pltpu.make_async_copy(bufs[b], acc.at[rstage.at[0]], ssems[b]).wait()

    for b in range(LOOK):
        _gather(b, b)

    def _iter(t, _):
        q0 = NBUF * t
        for b in range(NBUF):
            q = q0 + b
            n = (b + LOOK) % NBUF
            _wait_gather(b)

            @pl.when(q >= NBUF - LOOK)
            def _():
                _wait_scatter(n)

            @pl.when(q <= NCHUNK - LOOK - 1)
            def _():
                _gather(q + LOOK, n)

            _scatter(q, b)
        return 0
    lax.fori_loop(0, NCHUNK // NBUF, _iter, 0, unroll=False)
    for i in range(NBUF - LOOK):
        _wait_scatter((NCHUNK - (NBUF - LOOK) + i) % NBUF)
    plsc.subcore_barrier()

    pltpu.sync_copy(acc.at[pl.ds(s * 640, 640)],
                    out_hbm.at[c, pl.ds(s * 640, 640)])


def _make_spmm(d):
    def _body(x_hbm, ei3_hbm, out_hbm, cstage, rstage, *rest):
        bufs = rest[:NBUF]
        zbuf, acc = rest[NBUF], rest[NBUF + 1]
        gsems = rest[NBUF + 2:NBUF + 2 + NBUF]
        ssems = rest[NBUF + 2 + NBUF:NBUF + 2 + 2 * NBUF]
        stsem = rest[-1]
        _spmm_body(x_hbm, ei3_hbm, out_hbm, cstage, rstage, bufs, zbuf, acc,
                   gsems, ssems, stsem, d=d)

    return pl.kernel(
        _body,
        out_type=jax.ShapeDtypeStruct((NC, NPAD, d), jnp.float32),
        mesh=_MESH,
        scratch_types=[
            pltpu.VMEM((NCHUNK, CHUNK), jnp.int32),
            pltpu.VMEM((NCHUNK, CHUNK), jnp.int32),
        ] + [pltpu.VMEM((CHUNK, d), jnp.float32)] * NBUF + [
            pltpu.VMEM((640, d), jnp.float32),
            pltpu.VMEM_SHARED((NPAD, d), jnp.float32),
        ] + [pltpu.SemaphoreType.DMA] * (2 * NBUF + 1),
        compiler_params=pltpu.CompilerParams(
            needs_layout_passes=False, use_tc_tiling_on_sc=False),
    )


_spmm32 = _make_spmm(32)
_spmm16 = _make_spmm(16)


def _enc1_body(h_ref, w_ref, deg_ref, o_ref):
    d = deg_ref[...]
    norm = lax.rsqrt(d[0] + d[1])
    o_ref[...] = jnp.dot(h_ref[...], w_ref[...],
                         preferred_element_type=jnp.float32) * norm


_enc1_call = pl.pallas_call(
    _enc1_body,
    grid=(NPAD // 1024,),
    in_specs=[
        pl.BlockSpec((1024, 128), lambda i: (i, 0)),
        pl.BlockSpec((128, 32), lambda i: (0, 0)),
        pl.BlockSpec((2, 1024, 1), lambda i: (0, i, 0)),
    ],
    out_specs=pl.BlockSpec((1024, 32), lambda i: (i, 0)),
    out_shape=jax.ShapeDtypeStruct((NPAD, 32), jnp.float32),
)


def _enc2_body(p_ref, w_ref, deg_ref, o_ref):
    p = p_ref[...]
    hrelu = jnp.maximum(p[0] + p[1], 0.0)
    d = deg_ref[...]
    inv = 1.0 / (d[0] + d[1])
    o_ref[...] = jnp.dot(hrelu, w_ref[...],
                         preferred_element_type=jnp.float32) * inv


_enc2_call = pl.pallas_call(
    _enc2_body,
    grid=(NPAD // 1024,),
    in_specs=[
        pl.BlockSpec((2, 1024, 32), lambda i: (0, i, 0)),
        pl.BlockSpec((32, 16), lambda i: (0, 0)),
        pl.BlockSpec((2, 1024, 1), lambda i: (0, i, 0)),
    ],
    out_specs=pl.BlockSpec((1024, 16), lambda i: (i, 0)),
    out_shape=jax.ShapeDtypeStruct((NPAD, 16), jnp.float32),
)


def _dec_body(qi_ref, qj_ref, di_ref, dj_ref, o_ref):
    qi = qi_ref[...]
    di = di_ref[...]
    zi = (qi[0] + qi[1]) * lax.rsqrt(di[0] + di[1])
    qj = qj_ref[...]
    dj = dj_ref[...]
    zj = (qj[0] + qj[1]) * lax.rsqrt(dj[0] + dj[1])
    o_ref[...] = lax.dot_general(zi, zj, (((1,), (1,)), ((), ())),
                                 preferred_element_type=jnp.float32)


_BM = 2048
_BN = 2048
_dec_call = pl.pallas_call(
    _dec_body,
    grid=(NPAD // _BM, NPAD // _BN),
    in_specs=[
        pl.BlockSpec((2, _BM, 16), lambda i, j: (0, i, 0)),
        pl.BlockSpec((2, _BN, 16), lambda i, j: (0, j, 0)),
        pl.BlockSpec((2, _BM, 1), lambda i, j: (0, i, 0)),
        pl.BlockSpec((2, _BN, 1), lambda i, j: (0, j, 0)),
    ],
    out_specs=pl.BlockSpec((_BM, _BN), lambda i, j: (i, j)),
    out_shape=jax.ShapeDtypeStruct((N, N), jnp.float32),
    compiler_params=pltpu.CompilerParams(
        dimension_semantics=("parallel", "parallel")),
)


def kernel(h, edge_index, W0, W1):
    ei3 = edge_index.reshape(2, E // CHUNK, CHUNK)
    deg_p = _deg_call(edge_index)
    deg2 = deg_p.reshape(NC, NPAD, 1)
    x0 = _enc1_call(h, W0, deg2)
    P = _spmm32(x0, ei3)
    Q = _spmm16(_enc2_call(P, W1, deg2), ei3)
    return _dec_call(Q, Q, deg2, deg2)


# speedup vs baseline: 11.8685x; 1.0386x over previous
import functools

import jax
import jax.numpy as jnp
from jax import lax
from jax.experimental import pallas as pl
from jax.experimental.pallas import tpu as pltpu
from jax.experimental.pallas import tpu_sc as plsc

N = 10000
E = 320000
NPAD = 10240
NC = 2
NS = 16
NW = NC * NS
EPW = E // NW
CHUNK = 125

_MESH = plsc.VectorSubcoreMesh(core_axis_name="c", subcore_axis_name="s")

def _deg_body(ei_hbm, out_hbm, idxbuf, hist, tmp, accl, slots, sem):
    _ZERO16 = jnp.zeros((16,), jnp.float32)
    _ONES16 = jnp.ones((16,), jnp.float32)
    c = lax.axis_index("c")
    s = lax.axis_index("s")
    w = c * NS + s
    estart = pl.multiple_of(w * EPW, 8)

    def _z(i, _):
        hist[pl.ds(i * 16, 16)] = _ZERO16
        return 0
    lax.fori_loop(0, 640, _z, 0, unroll=False)

    def _outer(k, _):
        base = pl.multiple_of(estart + k * 2000, 8)
        pltpu.sync_copy(ei_hbm.at[0, pl.ds(base, 2000)], idxbuf)

        def _inner(j, _):
            idx = idxbuf[pl.ds(j * 16, 16)]
            plsc.addupdate_scatter(hist, [idx], _ONES16)
            return 0
        lax.fori_loop(0, 125, _inner, 0, unroll=False)
        return 0
    lax.fori_loop(0, 5, _outer, 0, unroll=False)

    pltpu.sync_copy(hist, slots.at[s])
    plsc.subcore_barrier()

    nbase = s * 640
    pltpu.sync_copy(slots.at[0, pl.ds(nbase, 640)], accl)

    def _red(j, _):
        pltpu.sync_copy(slots.at[j, pl.ds(nbase, 640)], tmp)

        def _add(i, _):
            accl[pl.ds(i * 16, 16)] = (
                accl[pl.ds(i * 16, 16)] + tmp[pl.ds(i * 16, 16)])
            return 0
        lax.fori_loop(0, 40, _add, 0, unroll=False)
        return 0
    lax.fori_loop(1, NS, _red, 0, unroll=False)

    pltpu.sync_copy(accl, out_hbm.at[c, pl.ds(nbase, 640)])


_deg_call = pl.kernel(
    _deg_body,
    out_type=jax.ShapeDtypeStruct((NC, NPAD), jnp.float32),
    mesh=_MESH,
    scratch_types=[
        pltpu.VMEM((2000,), jnp.int32),
        pltpu.VMEM((NPAD,), jnp.float32),
        pltpu.VMEM((640,), jnp.float32),
        pltpu.VMEM((640,), jnp.float32),
        pltpu.VMEM_SHARED((NS, NPAD), jnp.float32),
        pltpu.SemaphoreType.DMA,
    ],
    compiler_params=pltpu.CompilerParams(
        needs_layout_passes=False, use_tc_tiling_on_sc=False),
)


NCHUNK = EPW // CHUNK
NBUF = 10
LOOK = 4


def _spmm_body(x_hbm, ei3_hbm, out_hbm,
               cstage, rstage, bufs, zbuf, acc, gsems, ssems, stsem, *, d):
    _ZERO16 = jnp.zeros((16,), jnp.float32)
    c = lax.axis_index("c")
    s = lax.axis_index("s")
    w = c * NS + s
    cbase = w * NCHUNK

    pltpu.async_copy(ei3_hbm.at[1, pl.ds(cbase, NCHUNK)], cstage, stsem)
    pltpu.async_copy(ei3_hbm.at[0, pl.ds(cbase, NCHUNK)], rstage, stsem)

    def _z(i, _):
        for j in range(d // 16):
            zbuf[i, pl.ds(j * 16, 16)] = _ZERO16
        return 0
    lax.fori_loop(0, 640, _z, 0, unroll=False)
    pltpu.sync_copy(zbuf, acc.at[pl.ds(s * 640, 640)])
    pltpu.make_async_copy(ei3_hbm.at[1, pl.ds(cbase, NCHUNK)], cstage,
                          stsem).wait()
    pltpu.make_async_copy(ei3_hbm.at[0, pl.ds(cbase, NCHUNK)], rstage,
                          stsem).wait()
    plsc.subcore_barrier()

    def _gather(j, b):
        pltpu.async_copy(x_hbm.at[cstage.at[j]], bufs[b], gsems[b])

    def _wait_gather(b):
        pltpu.make_async_copy(x_hbm.at[cstage.at[0]], bufs[b], gsems[b]).wait()

    def _scatter(j, b):
        pltpu.async_copy(bufs[b], acc.at[rstage.at[j]], ssems[b], add=True)

    def _wait_scatter(b):
        pltpu.make_async_copy(bufs[b], acc.at[rstage.at[0]], ssems[b]).wait()

    for b in range(LOOK):
        _gather(b, b)

    def _iter(t, _):
        q0 = NBUF * t
        for b in range(NBUF):
            q = q0 + b
            n = (b + LOOK) % NBUF
            _wait_gather(b)

            @pl.when(q >= NBUF - LOOK)
            def _():
                _wait_scatter(n)

            @pl.when(q <= NCHUNK - LOOK - 1)
            def _():
                _gather(q + LOOK, n)

            _scatter(q, b)
        return 0
    lax.fori_loop(0, NCHUNK // NBUF, _iter, 0, unroll=False)
    for i in range(NBUF - LOOK):
        _wait_scatter((NCHUNK - (NBUF - LOOK) + i) % NBUF)
    plsc.subcore_barrier()

    pltpu.sync_copy(acc.at[pl.ds(s * 640, 640)],
                    out_hbm.at[c, pl.ds(s * 640, 640)])


def _make_spmm(d):
    def _body(x_hbm, ei3_hbm, out_hbm, cstage, rstage, *rest):
        bufs = rest[:NBUF]
        zbuf, acc = rest[NBUF], rest[NBUF + 1]
        gsems = rest[NBUF + 2:NBUF + 2 + NBUF]
        ssems = rest[NBUF + 2 + NBUF:NBUF + 2 + 2 * NBUF]
        stsem = rest[-1]
        _spmm_body(x_hbm, ei3_hbm, out_hbm, cstage, rstage, bufs, zbuf, acc,
                   gsems, ssems, stsem, d=d)

    return pl.kernel(
        _body,
        out_type=jax.ShapeDtypeStruct((NC, NPAD, d), jnp.float32),
        mesh=_MESH,
        scratch_types=[
            pltpu.VMEM((NCHUNK, CHUNK), jnp.int32),
            pltpu.VMEM((NCHUNK, CHUNK), jnp.int32),
        ] + [pltpu.VMEM((CHUNK, d), jnp.float32)] * NBUF + [
            pltpu.VMEM((640, d), jnp.float32),
            pltpu.VMEM_SHARED((NPAD, d), jnp.float32),
        ] + [pltpu.SemaphoreType.DMA] * (2 * NBUF + 1),
        compiler_params=pltpu.CompilerParams(
            needs_layout_passes=False, use_tc_tiling_on_sc=False),
    )


_spmm32 = _make_spmm(32)
_spmm16 = _make_spmm(16)


def _enc1_body(h_ref, w_ref, deg_ref, o_ref):
    d = deg_ref[...]
    norm = lax.rsqrt(d[0] + d[1])
    o_ref[...] = jnp.dot(h_ref[...], w_ref[...],
                         preferred_element_type=jnp.float32) * norm


_enc1_call = pl.pallas_call(
    _enc1_body,
    grid=(NPAD // 2048,),
    in_specs=[
        pl.BlockSpec((2048, 128), lambda i: (i, 0)),
        pl.BlockSpec((128, 32), lambda i: (0, 0)),
        pl.BlockSpec((2, 2048, 1), lambda i: (0, i, 0)),
    ],
    out_specs=pl.BlockSpec((2048, 32), lambda i: (i, 0)),
    out_shape=jax.ShapeDtypeStruct((NPAD, 32), jnp.float32),
)


def _enc2_body(p_ref, w_ref, deg_ref, o_ref):
    p = p_ref[...]
    hrelu = jnp.maximum(p[0] + p[1], 0.0)
    d = deg_ref[...]
    inv = 1.0 / (d[0] + d[1])
    o_ref[...] = jnp.dot(hrelu, w_ref[...],
                         preferred_element_type=jnp.float32) * inv


_enc2_call = pl.pallas_call(
    _enc2_body,
    grid=(NPAD // 2048,),
    in_specs=[
        pl.BlockSpec((2, 2048, 32), lambda i: (0, i, 0)),
        pl.BlockSpec((32, 16), lambda i: (0, 0)),
        pl.BlockSpec((2, 2048, 1), lambda i: (0, i, 0)),
    ],
    out_specs=pl.BlockSpec((2048, 16), lambda i: (i, 0)),
    out_shape=jax.ShapeDtypeStruct((NPAD, 16), jnp.float32),
)


def _dec_body(qi_ref, qj_ref, di_ref, dj_ref, o_ref):
    qi = qi_ref[...]
    di = di_ref[...]
    zi = (qi[0] + qi[1]) * lax.rsqrt(di[0] + di[1])
    qj = qj_ref[...]
    dj = dj_ref[...]
    zj = (qj[0] + qj[1]) * lax.rsqrt(dj[0] + dj[1])
    o_ref[...] = lax.dot_general(zi, zj, (((1,), (1,)), ((), ())),
                                 preferred_element_type=jnp.float32)


_BM = 2048
_BN = 2048
_dec_call = pl.pallas_call(
    _dec_body,
    grid=(NPAD // _BM, NPAD // _BN),
    in_specs=[
        pl.BlockSpec((2, _BM, 16), lambda i, j: (0, i, 0)),
        pl.BlockSpec((2, _BN, 16), lambda i, j: (0, j, 0)),
        pl.BlockSpec((2, _BM, 1), lambda i, j: (0, i, 0)),
        pl.BlockSpec((2, _BN, 1), lambda i, j: (0, j, 0)),
    ],
    out_specs=pl.BlockSpec((_BM, _BN), lambda i, j: (i, j)),
    out_shape=jax.ShapeDtypeStruct((N, N), jnp.float32),
    compiler_params=pltpu.CompilerParams(
        dimension_semantics=("parallel", "parallel")),
)


def kernel(h, edge_index, W0, W1):
    ei3 = edge_index.reshape(2, E // CHUNK, CHUNK)
    deg_p = _deg_call(edge_index)
    deg2 = deg_p.reshape(NC, NPAD, 1)
    x0 = _enc1_call(h, W0, deg2)
    P = _spmm32(x0, ei3)
    Q = _spmm16(_enc2_call(P, W1, deg2), ei3)
    return _dec_call(Q, Q, deg2, deg2)


# speedup vs baseline: 13.2996x; 1.1206x over previous
import functools

import jax
import jax.numpy as jnp
from jax import lax
from jax.experimental import pallas as pl
from jax.experimental.pallas import tpu as pltpu
from jax.experimental.pallas import tpu_sc as plsc

N = 10000
E = 320000
NPAD = 10240
NC = 2
NS = 16
NW = NC * NS
EPW = E // NW
CHUNK = 125

_MESH = plsc.VectorSubcoreMesh(core_axis_name="c", subcore_axis_name="s")

def _deg_body(ei_hbm, out_hbm, idxbuf, hist, tmp, accl, slots, sem):
    _ZERO16 = jnp.zeros((16,), jnp.float32)
    _ONES16 = jnp.ones((16,), jnp.float32)
    c = lax.axis_index("c")
    s = lax.axis_index("s")
    w = c * NS + s
    estart = pl.multiple_of(w * EPW, 8)

    def _z(i, _):
        hist[pl.ds(i * 16, 16)] = _ZERO16
        return 0
    lax.fori_loop(0, 640, _z, 0, unroll=False)

    def _outer(k, _):
        base = pl.multiple_of(estart + k * 2000, 8)
        pltpu.sync_copy(ei_hbm.at[0, pl.ds(base, 2000)], idxbuf)

        def _inner(j, _):
            idx = idxbuf[pl.ds(j * 16, 16)]
            plsc.addupdate_scatter(hist, [idx], _ONES16)
            return 0
        lax.fori_loop(0, 125, _inner, 0, unroll=False)
        return 0
    lax.fori_loop(0, 5, _outer, 0, unroll=False)

    pltpu.sync_copy(hist, slots.at[s])
    plsc.subcore_barrier()

    nbase = s * 640
    pltpu.sync_copy(slots.at[0, pl.ds(nbase, 640)], accl)

    def _red(j, _):
        pltpu.sync_copy(slots.at[j, pl.ds(nbase, 640)], tmp)

        def _add(i, _):
            accl[pl.ds(i * 16, 16)] = (
                accl[pl.ds(i * 16, 16)] + tmp[pl.ds(i * 16, 16)])
            return 0
        lax.fori_loop(0, 40, _add, 0, unroll=False)
        return 0
    lax.fori_loop(1, NS, _red, 0, unroll=False)

    pltpu.sync_copy(accl, out_hbm.at[c, pl.ds(nbase, 640)])


_deg_call = pl.kernel(
    _deg_body,
    out_type=jax.ShapeDtypeStruct((NC, NPAD), jnp.float32),
    mesh=_MESH,
    scratch_types=[
        pltpu.VMEM((2000,), jnp.int32),
        pltpu.VMEM((NPAD,), jnp.float32),
        pltpu.VMEM((640,), jnp.float32),
        pltpu.VMEM((640,), jnp.float32),
        pltpu.VMEM_SHARED((NS, NPAD), jnp.float32),
        pltpu.SemaphoreType.DMA,
    ],
    compiler_params=pltpu.CompilerParams(
        needs_layout_passes=False, use_tc_tiling_on_sc=False),
)


NCHUNK = EPW // CHUNK
NBUF = 10
LOOK = 4


def _spmm_body(x_hbm, ei3_hbm, out_hbm,
               cstage, rstage, bufs, zbuf, acc, gsems, ssems, stsem, *, d):
    _ZERO16 = jnp.zeros((16,), jnp.float32)
    c = lax.axis_index("c")
    s = lax.axis_index("s")
    w = c * NS + s
    cbase = w * NCHUNK

    pltpu.async_copy(ei3_hbm.at[1, pl.ds(cbase, NCHUNK)], cstage, stsem)
    pltpu.async_copy(ei3_hbm.at[0, pl.ds(cbase, NCHUNK)], rstage, stsem)

    def _z(i, _):
        for j in range(d // 16):
            zbuf[i, pl.ds(j * 16, 16)] = _ZERO16
        return 0
    lax.fori_loop(0, 640, _z, 0, unroll=False)
    pltpu.sync_copy(zbuf, acc.at[pl.ds(s * 640, 640)])
    pltpu.make_async_copy(ei3_hbm.at[1, pl.ds(cbase, NCHUNK)], cstage,
                          stsem).wait()
    pltpu.make_async_copy(ei3_hbm.at[0, pl.ds(cbase, NCHUNK)], rstage,
                          stsem).wait()
    plsc.subcore_barrier()

    def _gather(j, b):
        pltpu.async_copy(x_hbm.at[cstage.at[j]], bufs[b], gsems[b])

    def _wait_gather(b):
        pltpu.make_async_copy(x_hbm.at[cstage.at[0]], bufs[b], gsems[b]).wait()

    def _scatter(j, b):
        pltpu.async_copy(bufs[b], acc.at[rstage.at[j]], ssems[b], add=True)

    def _wait_scatter(b):
        pltpu.make_async_copy(bufs[b], acc.at[rstage.at[0]], ssems[b]).wait()

    for b in range(LOOK):
        _gather(b, b)

    def _iter(t, _):
        q0 = NBUF * t
        for b in range(NBUF):
            q = q0 + b
            n = (b + LOOK) % NBUF
            _wait_gather(b)

            @pl.when(q >= NBUF - LOOK)
            def _():
                _wait_scatter(n)

            @pl.when(q <= NCHUNK - LOOK - 1)
            def _():
                _gather(q + LOOK, n)

            _scatter(q, b)
        return 0
    lax.fori_loop(0, NCHUNK // NBUF, _iter, 0, unroll=False)
    for i in range(NBUF - LOOK):
        _wait_scatter((NCHUNK - (NBUF - LOOK) + i) % NBUF)
    plsc.subcore_barrier()

    pltpu.sync_copy(acc.at[pl.ds(s * 640, 640)],
                    out_hbm.at[c, pl.ds(s * 640, 640)])


def _make_spmm(d):
    def _body(x_hbm, ei3_hbm, out_hbm, cstage, rstage, *rest):
        bufs = rest[:NBUF]
        zbuf, acc = rest[NBUF], rest[NBUF + 1]
        gsems = rest[NBUF + 2:NBUF + 2 + NBUF]
        ssems = rest[NBUF + 2 + NBUF:NBUF + 2 + 2 * NBUF]
        stsem = rest[-1]
        _spmm_body(x_hbm, ei3_hbm, out_hbm, cstage, rstage, bufs, zbuf, acc,
                   gsems, ssems, stsem, d=d)

    return pl.kernel(
        _body,
        out_type=jax.ShapeDtypeStruct((NC, NPAD, d), jnp.float32),
        mesh=_MESH,
        scratch_types=[
            pltpu.VMEM((NCHUNK, CHUNK), jnp.int32),
            pltpu.VMEM((NCHUNK, CHUNK), jnp.int32),
        ] + [pltpu.VMEM((CHUNK, d), jnp.float32)] * NBUF + [
            pltpu.VMEM((640, d), jnp.float32),
            pltpu.VMEM_SHARED((NPAD, d), jnp.float32),
        ] + [pltpu.SemaphoreType.DMA] * (2 * NBUF + 1),
        compiler_params=pltpu.CompilerParams(
            needs_layout_passes=False, use_tc_tiling_on_sc=False),
    )


_spmm32 = _make_spmm(32)
_spmm16 = _make_spmm(16)


def _enc1_body(h_ref, w_ref, deg_ref, o_ref):
    d = deg_ref[...]
    norm = lax.rsqrt(d[0] + d[1]).reshape(d.shape[1], 1)
    o_ref[...] = jnp.dot(h_ref[...], w_ref[...],
                         preferred_element_type=jnp.float32) * norm


_enc1_call = pl.pallas_call(
    _enc1_body,
    grid=(NPAD // 2048,),
    in_specs=[
        pl.BlockSpec((2048, 128), lambda i: (i, 0)),
        pl.BlockSpec((128, 32), lambda i: (0, 0)),
        pl.BlockSpec((2, 2048), lambda i: (0, i)),
    ],
    out_specs=pl.BlockSpec((2048, 32), lambda i: (i, 0)),
    out_shape=jax.ShapeDtypeStruct((NPAD, 32), jnp.float32),
)


def _enc2_body(p_ref, w_ref, deg_ref, o_ref):
    p = p_ref[...]
    hrelu = jnp.maximum(p[0] + p[1], 0.0)
    d = deg_ref[...]
    inv = (1.0 / (d[0] + d[1])).reshape(d.shape[1], 1)
    o_ref[...] = jnp.dot(hrelu, w_ref[...],
                         preferred_element_type=jnp.float32) * inv


_enc2_call = pl.pallas_call(
    _enc2_body,
    grid=(NPAD // 2048,),
    in_specs=[
        pl.BlockSpec((2, 2048, 32), lambda i: (0, i, 0)),
        pl.BlockSpec((32, 16), lambda i: (0, 0)),
        pl.BlockSpec((2, 2048), lambda i: (0, i)),
    ],
    out_specs=pl.BlockSpec((2048, 16), lambda i: (i, 0)),
    out_shape=jax.ShapeDtypeStruct((NPAD, 16), jnp.float32),
)


def _dec_body(qi_ref, qj_ref, di_ref, dj_ref, o_ref):
    qi = qi_ref[...]
    di = di_ref[...]
    zi = (qi[0] + qi[1]) * lax.rsqrt(di[0] + di[1]).reshape(di.shape[1], 1)
    qj = qj_ref[...]
    dj = dj_ref[...]
    zj = (qj[0] + qj[1]) * lax.rsqrt(dj[0] + dj[1]).reshape(dj.shape[1], 1)
    o_ref[...] = lax.dot_general(zi, zj, (((1,), (1,)), ((), ())),
                                 preferred_element_type=jnp.float32)


_BM = 2048
_BN = 2048
_dec_call = pl.pallas_call(
    _dec_body,
    grid=(NPAD // _BM, NPAD // _BN),
    in_specs=[
        pl.BlockSpec((2, _BM, 16), lambda i, j: (0, i, 0)),
        pl.BlockSpec((2, _BN, 16), lambda i, j: (0, j, 0)),
        pl.BlockSpec((2, _BM), lambda i, j: (0, i)),
        pl.BlockSpec((2, _BN), lambda i, j: (0, j)),
    ],
    out_specs=pl.BlockSpec((_BM, _BN), lambda i, j: (i, j)),
    out_shape=jax.ShapeDtypeStruct((N, N), jnp.float32),
    compiler_params=pltpu.CompilerParams(
        dimension_semantics=("parallel", "parallel")),
)


def kernel(h, edge_index, W0, W1):
    ei3 = edge_index.reshape(2, E // CHUNK, CHUNK)
    deg_p = _deg_call(edge_index)
    x0 = _enc1_call(h, W0, deg_p)
    P = _spmm32(x0, ei3)
    Q = _spmm16(_enc2_call(P, W1, deg_p), ei3)
    return _dec_call(Q, Q, deg_p, deg_p)
